# Initial kernel scaffold; baseline (speedup 1.0000x reference)
import jax
import jax.numpy as jnp
from jax.experimental import pallas as pl


def kernel(x, h, edge_index, local_frames, batch, W1, b1, g1, be1, W2, b2, g2, be2, Wg, W3, b3, g3, be3, W4, b4, g4, be4, W5, b5, g5, be5, W6, b6):
    raise NotImplementedError("write your pallas kernel here")


# speedup vs baseline: 1.0025x; 1.0025x over previous
import functools

import jax
import jax.numpy as jnp
from jax import lax
from jax.experimental import pallas as pl
from jax.experimental.pallas import tpu as pltpu

_EPS = 1e-5


def _ln(t, g, b):
    mu = jnp.mean(t, axis=-1, keepdims=True)
    var = jnp.mean((t - mu) ** 2, axis=-1, keepdims=True)
    return (t - mu) * lax.rsqrt(var + _EPS) * g + b


def _silu(t):
    return t * jax.nn.sigmoid(t)


def _k1_body(h_ref, wa_ref, wb_ref, a_ref, b_ref):
    h = h_ref[...]
    a_ref[...] = jnp.dot(h, wa_ref[...], preferred_element_type=jnp.float32)
    b_ref[...] = jnp.dot(h, wb_ref[...], preferred_element_type=jnp.float32)


def _node_proj(h, wa, wb, bn=1000):
    n, d = h.shape
    grid = (n // bn,)
    return pl.pallas_call(
        _k1_body,
        grid=grid,
        in_specs=[
            pl.BlockSpec((bn, d), lambda i: (i, 0)),
            pl.BlockSpec((d, d), lambda i: (0, 0)),
            pl.BlockSpec((d, d), lambda i: (0, 0)),
        ],
        out_specs=[
            pl.BlockSpec((bn, d), lambda i: (i, 0)),
            pl.BlockSpec((bn, d), lambda i: (i, 0)),
        ],
        out_shape=[
            jax.ShapeDtypeStruct((n, d), jnp.float32),
            jax.ShapeDtypeStruct((n, d), jnp.float32),
        ],
    )(h, wa, wb)


def _k3_body(p_ref, r2_ref, pot_ref, rx_ref, ry_ref, rz_ref,
             w1c_ref, w1d_ref, b1_ref, g1_ref, be1_ref,
             w2_ref, b2_ref, g2_ref, be2_ref, wg_ref,
             w5_ref, b5_ref, g5_ref, be5_ref, w6_ref, b6_ref,
             mij_ref, dx_ref):
    r2 = r2_ref[...]
    pot = pot_ref[...]
    pre1 = p_ref[...] + r2 * w1c_ref[...] + pot * w1d_ref[...] + b1_ref[...]
    u = _silu(_ln(pre1, g1_ref[...], be1_ref[...]))
    m = jnp.dot(u, w2_ref[...], preferred_element_type=jnp.float32) + b2_ref[...]
    m = _silu(_ln(m, g2_ref[...], be2_ref[...]))
    gate = jax.nn.sigmoid(jnp.dot(m, wg_ref[...], preferred_element_type=jnp.float32))
    mij = m * gate
    mij_ref[...] = mij
    t = jnp.dot(mij, w5_ref[...], preferred_element_type=jnp.float32) + b5_ref[...]
    t = _silu(_ln(t, g5_ref[...], be5_ref[...]))
    w = jax.nn.sigmoid(jnp.dot(t, w6_ref[...], preferred_element_type=jnp.float32)
                       + b6_ref[...])
    zero = jnp.zeros_like(w)
    dx_ref[...] = jnp.concatenate(
        [rx_ref[...] * w, ry_ref[...] * w, rz_ref[...] * w, zero], axis=-1)


def _edge_mlp(p, r2, pot, rx, ry, rz,
              w1c, w1d, b1, g1, be1, w2, b2, g2, be2, wg,
              w5, b5, g5, be5, w6, b6, be_blk=2000):
    e, d = p.shape
    grid = (e // be_blk,)
    col1 = lambda i: (i, 0)
    full = lambda i: (0, 0)
    spec_e1 = pl.BlockSpec((be_blk, 1), col1)
    spec_dd = pl.BlockSpec((d, d), full)
    spec_1d = pl.BlockSpec((1, d), full)
    spec_d1 = pl.BlockSpec((d, 1), full)
    spec_11 = pl.BlockSpec((1, 1), full)
    return pl.pallas_call(
        _k3_body,
        grid=grid,
        in_specs=[
            pl.BlockSpec((be_blk, d), col1),
            spec_e1, spec_e1, spec_e1, spec_e1, spec_e1,
            spec_1d, spec_1d, spec_1d, spec_1d, spec_1d,
            spec_dd, spec_1d, spec_1d, spec_1d, spec_d1,
            spec_dd, spec_1d, spec_1d, spec_1d, spec_d1, spec_11,
        ],
        out_specs=[
            pl.BlockSpec((be_blk, d), col1),
            pl.BlockSpec((be_blk, 4), col1),
        ],
        out_shape=[
            jax.ShapeDtypeStruct((e, d), jnp.float32),
            jax.ShapeDtypeStruct((e, 4), jnp.float32),
        ],
    )(p, r2, pot, rx, ry, rz, w1c, w1d, b1, g1, be1,
      w2, b2, g2, be2, wg, w5, b5, g5, be5, w6, b6)


def _k5_body(h_ref, mi_ref, x4_ref, dxs_ref,
             w3a_ref, w3b_ref, b3_ref, g3_ref, be3_ref,
             w4_ref, b4_ref, g4_ref, be4_ref,
             hn_ref, xn_ref):
    h = h_ref[...]
    mi = mi_ref[...]
    q = (jnp.dot(h, w3a_ref[...], preferred_element_type=jnp.float32)
         + jnp.dot(mi, w3b_ref[...], preferred_element_type=jnp.float32)
         + b3_ref[...])
    q = _silu(_ln(q, g3_ref[...], be3_ref[...]))
    ph = jnp.dot(q, w4_ref[...], preferred_element_type=jnp.float32) + b4_ref[...]
    ph = _ln(ph, g4_ref[...], be4_ref[...])
    hn_ref[...] = h + ph
    xn_ref[...] = x4_ref[...] + dxs_ref[...]


def _node_mlp(h, mi, x4, dxs, w3a, w3b, b3, g3, be3, w4, b4, g4, be4, bn=1000):
    n, d = h.shape
    grid = (n // bn,)
    col1 = lambda i: (i, 0)
    full = lambda i: (0, 0)
    spec_dd = pl.BlockSpec((d, d), full)
    spec_1d = pl.BlockSpec((1, d), full)
    return pl.pallas_call(
        _k5_body,
        grid=grid,
        in_specs=[
            pl.BlockSpec((bn, d), col1),
            pl.BlockSpec((bn, d), col1),
            pl.BlockSpec((bn, 4), col1),
            pl.BlockSpec((bn, 4), col1),
            spec_dd, spec_dd, spec_1d, spec_1d, spec_1d,
            spec_dd, spec_1d, spec_1d, spec_1d,
        ],
        out_specs=[
            pl.BlockSpec((bn, d), col1),
            pl.BlockSpec((bn, 4), col1),
        ],
        out_shape=[
            jax.ShapeDtypeStruct((n, d), jnp.float32),
            jax.ShapeDtypeStruct((n, 4), jnp.float32),
        ],
    )(h, mi, x4, dxs, w3a, w3b, b3, g3, be3, w4, b4, g4, be4)


def kernel(x, h, edge_index, local_frames, batch,
           W1, b1, g1, be1, W2, b2, g2, be2, Wg,
           W3, b3, g3, be3, W4, b4, g4, be4,
           W5, b5, g5, be5, W6, b6):
    n, din = h.shape
    e = edge_index.shape[1]
    row = edge_index[0]
    col = edge_index[1]

    w1a = W1[:din]
    w1b = W1[din:2 * din]
    w1c = W1[2 * din:2 * din + 1]
    w1d = W1[2 * din + 1:2 * din + 2]

    A, B = _node_proj(h, w1a, w1b)

    p = A[col] + B[row]
    r_ij = x[col] - x[row]
    r2 = jnp.sum(r_ij ** 2, axis=-1, keepdims=True)
    pot = 1.0 / (r2 + 1e-6)
    rx = r_ij[:, 0:1]
    ry = r_ij[:, 1:2]
    rz = r_ij[:, 2:3]

    mij, dx4 = _edge_mlp(
        p, r2, pot, rx, ry, rz,
        w1c, w1d, b1.reshape(1, -1), g1.reshape(1, -1), be1.reshape(1, -1),
        W2, b2.reshape(1, -1), g2.reshape(1, -1), be2.reshape(1, -1), Wg,
        W5, b5.reshape(1, -1), g5.reshape(1, -1), be5.reshape(1, -1),
        W6, b6.reshape(1, 1))

    mi = jnp.zeros((n, din), jnp.float32).at[col].add(mij)
    dxs = jnp.zeros((n, 4), jnp.float32).at[row].add(dx4)

    x4 = jnp.concatenate([x, jnp.zeros((n, 1), jnp.float32)], axis=-1)
    hn, xn4 = _node_mlp(
        h, mi, x4, dxs,
        W3[:din], W3[din:], b3.reshape(1, -1), g3.reshape(1, -1),
        be3.reshape(1, -1), W4, b4.reshape(1, -1), g4.reshape(1, -1),
        be4.reshape(1, -1))

    return (xn4[:, :3], hn)


# speedup vs baseline: 2.0442x; 2.0390x over previous
import functools

import jax
import jax.numpy as jnp
from jax import lax
from jax.experimental import pallas as pl
from jax.experimental.pallas import tpu as pltpu
from jax.experimental.pallas import tpu_sc as plsc

_EPS = 1e-5
_NC, _NS, _L = 2, 16, 16
_NW = _NC * _NS


def _ln(t, g, b):
    mu = jnp.mean(t, axis=-1, keepdims=True)
    var = jnp.mean((t - mu) ** 2, axis=-1, keepdims=True)
    return (t - mu) * lax.rsqrt(var + _EPS) * g + b


def _silu(t):
    return t * jax.nn.sigmoid(t)


def _k1_body(h_ref, wa_ref, wb_ref, a_ref, b_ref):
    h = h_ref[...]
    a_ref[...] = jnp.dot(h, wa_ref[...], preferred_element_type=jnp.float32)
    b_ref[...] = jnp.dot(h, wb_ref[...], preferred_element_type=jnp.float32)


def _node_proj(h, wa, wb, bn=1000):
    n, d = h.shape
    grid = (n // bn,)
    return pl.pallas_call(
        _k1_body,
        grid=grid,
        in_specs=[
            pl.BlockSpec((bn, d), lambda i: (i, 0)),
            pl.BlockSpec((d, d), lambda i: (0, 0)),
            pl.BlockSpec((d, d), lambda i: (0, 0)),
        ],
        out_specs=[
            pl.BlockSpec((bn, d), lambda i: (i, 0)),
            pl.BlockSpec((bn, d), lambda i: (i, 0)),
        ],
        out_shape=[
            jax.ShapeDtypeStruct((n, d), jnp.float32),
            jax.ShapeDtypeStruct((n, d), jnp.float32),
        ],
    )(h, wa, wb)


def _sc_gather(A, B, x4flat, col, row, chunk=80):
    n, d = A.shape
    e = col.shape[0]
    ept = e // _NW
    nchunks = ept // chunk
    mesh = plsc.VectorSubcoreMesh(core_axis_name="c", subcore_axis_name="s")
    fvec = jax.ShapeDtypeStruct((e,), jnp.float32)

    @functools.partial(
        pl.kernel, mesh=mesh,
        out_type=(
            jax.ShapeDtypeStruct((e, d), jnp.float32),
            fvec, fvec, fvec, fvec, fvec,
        ),
        compiler_params=pltpu.CompilerParams(needs_layout_passes=False),
        scratch_types=[
            pltpu.VMEM((chunk,), jnp.int32),
            pltpu.VMEM((chunk,), jnp.int32),
            pltpu.VMEM((chunk, d), jnp.float32),
            pltpu.VMEM((chunk, d), jnp.float32),
            pltpu.VMEM((chunk, d), jnp.float32),
            pltpu.VMEM((n * 4,), jnp.float32),
            pltpu.VMEM((chunk,), jnp.float32),
            pltpu.VMEM((chunk,), jnp.float32),
            pltpu.VMEM((chunk,), jnp.float32),
            pltpu.VMEM((chunk,), jnp.float32),
            pltpu.VMEM((chunk,), jnp.float32),
            pltpu.SemaphoreType.DMA,
            pltpu.SemaphoreType.DMA,
        ],
    )
    def k(a_hbm, b_hbm, x_hbm, col_hbm, row_hbm,
          p_hbm, r2_hbm, pot_hbm, rx_hbm, ry_hbm, rz_hbm,
          idxc, idxr, abuf, bbuf, pbuf, xtab, r2b, potb, rxb, ryb, rzb,
          sem1, sem2):
        wid = lax.axis_index("s") * _NC + lax.axis_index("c")
        pltpu.sync_copy(x_hbm, xtab)

        def chunk_body(ci, _):
            base = wid * ept + ci * chunk
            pltpu.sync_copy(col_hbm.at[pl.ds(base, chunk)], idxc)
            pltpu.sync_copy(row_hbm.at[pl.ds(base, chunk)], idxr)
            ca = pltpu.async_copy(a_hbm.at[idxc], abuf, sem1)
            cb = pltpu.async_copy(b_hbm.at[idxr], bbuf, sem2)
            ca.wait()
            cb.wait()

            def add_body(i, _):
                for j in range(d // _L):
                    s = pl.ds(j * _L, _L)
                    pbuf[i, s] = abuf[i, s] + bbuf[i, s]
                return 0
            lax.fori_loop(0, chunk, add_body, 0)

            for v in range(chunk // _L):
                s = pl.ds(v * _L, _L)
                cv = idxc[s] * 4
                rv = idxr[s] * 4
                dxc = plsc.load_gather(xtab, [cv]) - plsc.load_gather(xtab, [rv])
                dyc = (plsc.load_gather(xtab, [cv + 1])
                       - plsc.load_gather(xtab, [rv + 1]))
                dzc = (plsc.load_gather(xtab, [cv + 2])
                       - plsc.load_gather(xtab, [rv + 2]))
                r2v = dxc * dxc + dyc * dyc + dzc * dzc
                rxb[s] = dxc
                ryb[s] = dyc
                rzb[s] = dzc
                r2b[s] = r2v
                potb[s] = 1.0 / (r2v + 1e-6)

            pltpu.sync_copy(pbuf, p_hbm.at[pl.ds(base, chunk)])
            pltpu.sync_copy(r2b, r2_hbm.at[pl.ds(base, chunk)])
            pltpu.sync_copy(potb, pot_hbm.at[pl.ds(base, chunk)])
            pltpu.sync_copy(rxb, rx_hbm.at[pl.ds(base, chunk)])
            pltpu.sync_copy(ryb, ry_hbm.at[pl.ds(base, chunk)])
            pltpu.sync_copy(rzb, rz_hbm.at[pl.ds(base, chunk)])
            return 0

        lax.fori_loop(0, nchunks, chunk_body, 0)

    return k(A, B, x4flat, col, row)


def _k3_body(p_ref, r2_ref, pot_ref, rx_ref, ry_ref, rz_ref,
             w1c_ref, w1d_ref, b1_ref, g1_ref, be1_ref,
             w2_ref, b2_ref, g2_ref, be2_ref, wg_ref,
             w5_ref, b5_ref, g5_ref, be5_ref, w6_ref, b6_ref,
             mij_ref, dx_ref):
    r2 = r2_ref[...]
    pot = pot_ref[...]
    pre1 = p_ref[...] + r2 * w1c_ref[...] + pot * w1d_ref[...] + b1_ref[...]
    u = _silu(_ln(pre1, g1_ref[...], be1_ref[...]))
    m = jnp.dot(u, w2_ref[...], preferred_element_type=jnp.float32) + b2_ref[...]
    m = _silu(_ln(m, g2_ref[...], be2_ref[...]))
    gate = jax.nn.sigmoid(jnp.dot(m, wg_ref[...], preferred_element_type=jnp.float32))
    mij = m * gate
    mij_ref[...] = mij
    t = jnp.dot(mij, w5_ref[...], preferred_element_type=jnp.float32) + b5_ref[...]
    t = _silu(_ln(t, g5_ref[...], be5_ref[...]))
    w = jax.nn.sigmoid(jnp.dot(t, w6_ref[...], preferred_element_type=jnp.float32)
                       + b6_ref[...])
    zero = jnp.zeros_like(w)
    dx_ref[...] = jnp.concatenate(
        [rx_ref[...] * w, ry_ref[...] * w, rz_ref[...] * w, zero], axis=-1)


def _edge_mlp(p, r2, pot, rx, ry, rz,
              w1c, w1d, b1, g1, be1, w2, b2, g2, be2, wg,
              w5, b5, g5, be5, w6, b6, be_blk=2000):
    e, d = p.shape
    grid = (e // be_blk,)
    col1 = lambda i: (i, 0)
    full = lambda i: (0, 0)
    spec_e1 = pl.BlockSpec((be_blk, 1), col1)
    spec_dd = pl.BlockSpec((d, d), full)
    spec_1d = pl.BlockSpec((1, d), full)
    spec_d1 = pl.BlockSpec((d, 1), full)
    spec_11 = pl.BlockSpec((1, 1), full)
    return pl.pallas_call(
        _k3_body,
        grid=grid,
        in_specs=[
            pl.BlockSpec((be_blk, d), col1),
            spec_e1, spec_e1, spec_e1, spec_e1, spec_e1,
            spec_1d, spec_1d, spec_1d, spec_1d, spec_1d,
            spec_dd, spec_1d, spec_1d, spec_1d, spec_d1,
            spec_dd, spec_1d, spec_1d, spec_1d, spec_d1, spec_11,
        ],
        out_specs=[
            pl.BlockSpec((be_blk, d), col1),
            pl.BlockSpec((be_blk, 4), col1),
        ],
        out_shape=[
            jax.ShapeDtypeStruct((e, d), jnp.float32),
            jax.ShapeDtypeStruct((e, 4), jnp.float32),
        ],
    )(p, r2, pot, rx, ry, rz, w1c, w1d, b1, g1, be1,
      w2, b2, g2, be2, wg, w5, b5, g5, be5, w6, b6)


def _sc_scatter(mij, col, npad, chunk=80):
    e, d = mij.shape
    ept = e // _NW
    nchunks = ept // chunk
    npt = npad // _NS
    mesh = plsc.VectorSubcoreMesh(core_axis_name="c", subcore_axis_name="s")

    @functools.partial(
        pl.kernel, mesh=mesh,
        out_type=jax.ShapeDtypeStruct((_NC, npad, d), jnp.float32),
        compiler_params=pltpu.CompilerParams(needs_layout_passes=False),
        scratch_types=[
            pltpu.VMEM((chunk,), jnp.int32),
            pltpu.VMEM((chunk, d), jnp.float32),
            pltpu.VMEM_SHARED((npad, d), jnp.float32),
        ],
    )
    def k(m_hbm, col_hbm, zm_hbm, om_hbm, idxc, mbuf, sm):
        cid = lax.axis_index("c")
        sid = lax.axis_index("s")
        wid = sid * _NC + cid
        nbase = sid * npt

        pltpu.sync_copy(zm_hbm.at[pl.ds(nbase, npt)], sm.at[pl.ds(nbase, npt)])
        plsc.subcore_barrier()

        def chunk_body(ci, _):
            base = wid * ept + ci * chunk
            pltpu.sync_copy(col_hbm.at[pl.ds(base, chunk)], idxc)
            pltpu.sync_copy(m_hbm.at[pl.ds(base, chunk)], mbuf)
            pltpu.sync_copy(mbuf, sm.at[idxc], add=True)
            return 0

        lax.fori_loop(0, nchunks, chunk_body, 0)
        plsc.subcore_barrier()
        pltpu.sync_copy(sm.at[pl.ds(nbase, npt)], om_hbm.at[cid, pl.ds(nbase, npt)])

    return k(mij, col, jnp.zeros((npad, d), jnp.float32))


def _k5_body(h_ref, mi0_ref, mi1_ref, x4_ref, dx0_ref, dx1_ref,
             w3a_ref, w3b_ref, b3_ref, g3_ref, be3_ref,
             w4_ref, b4_ref, g4_ref, be4_ref,
             hn_ref, xn_ref):
    h = h_ref[...]
    mi = mi0_ref[...] + mi1_ref[...]
    q = (jnp.dot(h, w3a_ref[...], preferred_element_type=jnp.float32)
         + jnp.dot(mi, w3b_ref[...], preferred_element_type=jnp.float32)
         + b3_ref[...])
    q = _silu(_ln(q, g3_ref[...], be3_ref[...]))
    ph = jnp.dot(q, w4_ref[...], preferred_element_type=jnp.float32) + b4_ref[...]
    ph = _ln(ph, g4_ref[...], be4_ref[...])
    hn_ref[...] = h + ph
    xn_ref[...] = x4_ref[...] + dx0_ref[...] + dx1_ref[...]


def _node_mlp(h, mi0, mi1, x4, dx0, dx1,
              w3a, w3b, b3, g3, be3, w4, b4, g4, be4, bn=1000):
    n, d = h.shape
    grid = (n // bn,)
    col1 = lambda i: (i, 0)
    full = lambda i: (0, 0)
    spec_dd = pl.BlockSpec((d, d), full)
    spec_1d = pl.BlockSpec((1, d), full)
    return pl.pallas_call(
        _k5_body,
        grid=grid,
        in_specs=[
            pl.BlockSpec((bn, d), col1),
            pl.BlockSpec((bn, d), col1),
            pl.BlockSpec((bn, d), col1),
            pl.BlockSpec((bn, 4), col1),
            pl.BlockSpec((bn, 4), col1),
            pl.BlockSpec((bn, 4), col1),
            spec_dd, spec_dd, spec_1d, spec_1d, spec_1d,
            spec_dd, spec_1d, spec_1d, spec_1d,
        ],
        out_specs=[
            pl.BlockSpec((bn, d), col1),
            pl.BlockSpec((bn, 4), col1),
        ],
        out_shape=[
            jax.ShapeDtypeStruct((n, d), jnp.float32),
            jax.ShapeDtypeStruct((n, 4), jnp.float32),
        ],
    )(h, mi0, mi1, x4, dx0, dx1, w3a, w3b, b3, g3, be3, w4, b4, g4, be4)


def kernel(x, h, edge_index, local_frames, batch,
           W1, b1, g1, be1, W2, b2, g2, be2, Wg,
           W3, b3, g3, be3, W4, b4, g4, be4,
           W5, b5, g5, be5, W6, b6):
    n, din = h.shape
    e = edge_index.shape[1]
    row = edge_index[0]
    col = edge_index[1]

    w1a = W1[:din]
    w1b = W1[din:2 * din]
    w1c = W1[2 * din:2 * din + 1]
    w1d = W1[2 * din + 1:2 * din + 2]

    A, B = _node_proj(h, w1a, w1b)

    x4 = jnp.concatenate([x, jnp.zeros((n, 1), jnp.float32)], axis=-1)
    p, r2, pot, rx, ry, rz = _sc_gather(A, B, x4.reshape(-1), col, row)
    r2 = r2.reshape(e, 1)
    pot = pot.reshape(e, 1)
    rx = rx.reshape(e, 1)
    ry = ry.reshape(e, 1)
    rz = rz.reshape(e, 1)

    mij, dx4 = _edge_mlp(
        p, r2, pot, rx, ry, rz,
        w1c, w1d, b1.reshape(1, -1), g1.reshape(1, -1), be1.reshape(1, -1),
        W2, b2.reshape(1, -1), g2.reshape(1, -1), be2.reshape(1, -1), Wg,
        W5, b5.reshape(1, -1), g5.reshape(1, -1), be5.reshape(1, -1),
        W6, b6.reshape(1, 1))

    npad = ((n + 8 * _NS - 1) // (8 * _NS)) * (8 * _NS)
    om = _sc_scatter(mij, col, npad)
    dxs = jnp.zeros((n, 4), jnp.float32).at[row].add(dx4)
    zx4 = jnp.zeros((n, 4), jnp.float32)

    hn, xn4 = _node_mlp(
        h, om[0, :n], om[1, :n], x4, dxs, zx4,
        W3[:din], W3[din:], b3.reshape(1, -1), g3.reshape(1, -1),
        be3.reshape(1, -1), W4, b4.reshape(1, -1), g4.reshape(1, -1),
        be4.reshape(1, -1))

    return (xn4[:, :3], hn)


# speedup vs baseline: 2.3908x; 1.1696x over previous
import functools

import jax
import jax.numpy as jnp
from jax import lax
from jax.experimental import pallas as pl
from jax.experimental.pallas import tpu as pltpu
from jax.experimental.pallas import tpu_sc as plsc

_EPS = 1e-5
_NC, _NS, _L = 2, 16, 16
_NW = _NC * _NS


def _ln(t, g, b):
    mu = jnp.mean(t, axis=-1, keepdims=True)
    var = jnp.mean((t - mu) ** 2, axis=-1, keepdims=True)
    return (t - mu) * lax.rsqrt(var + _EPS) * g + b


def _silu(t):
    return t * jax.nn.sigmoid(t)


def _k1_body(h_ref, wa_ref, wb_ref, a_ref, b_ref):
    h = h_ref[...]
    a_ref[...] = jnp.dot(h, wa_ref[...], preferred_element_type=jnp.float32)
    b_ref[...] = jnp.dot(h, wb_ref[...], preferred_element_type=jnp.float32)


def _node_proj(h, wa, wb, bn=1000):
    n, d = h.shape
    grid = (n // bn,)
    return pl.pallas_call(
        _k1_body,
        grid=grid,
        in_specs=[
            pl.BlockSpec((bn, d), lambda i: (i, 0)),
            pl.BlockSpec((d, d), lambda i: (0, 0)),
            pl.BlockSpec((d, d), lambda i: (0, 0)),
        ],
        out_specs=[
            pl.BlockSpec((bn, d), lambda i: (i, 0)),
            pl.BlockSpec((bn, d), lambda i: (i, 0)),
        ],
        out_shape=[
            jax.ShapeDtypeStruct((n, d), jnp.float32),
            jax.ShapeDtypeStruct((n, d), jnp.float32),
        ],
    )(h, wa, wb)


def _sc_gather(A, B, x4flat, col, row, chunk=80):
    n, d = A.shape
    e = col.shape[0]
    ept = e // _NW
    nchunks = ept // chunk
    mesh = plsc.VectorSubcoreMesh(core_axis_name="c", subcore_axis_name="s")
    fvec = jax.ShapeDtypeStruct((e,), jnp.float32)

    @functools.partial(
        pl.kernel, mesh=mesh,
        out_type=(
            jax.ShapeDtypeStruct((e, d), jnp.float32),
            fvec, fvec, fvec, fvec, fvec,
        ),
        compiler_params=pltpu.CompilerParams(needs_layout_passes=False),
        scratch_types=[
            pltpu.VMEM((chunk,), jnp.int32),
            pltpu.VMEM((chunk,), jnp.int32),
            pltpu.VMEM((chunk, d), jnp.float32),
            pltpu.VMEM((chunk, d), jnp.float32),
            pltpu.VMEM((chunk, d), jnp.float32),
            pltpu.VMEM((n * 4,), jnp.float32),
            pltpu.VMEM((chunk,), jnp.float32),
            pltpu.VMEM((chunk,), jnp.float32),
            pltpu.VMEM((chunk,), jnp.float32),
            pltpu.VMEM((chunk,), jnp.float32),
            pltpu.VMEM((chunk,), jnp.float32),
            pltpu.SemaphoreType.DMA,
            pltpu.SemaphoreType.DMA,
        ],
    )
    def k(a_hbm, b_hbm, x_hbm, col_hbm, row_hbm,
          p_hbm, r2_hbm, pot_hbm, rx_hbm, ry_hbm, rz_hbm,
          idxc, idxr, abuf, bbuf, pbuf, xtab, r2b, potb, rxb, ryb, rzb,
          sem1, sem2):
        wid = lax.axis_index("s") * _NC + lax.axis_index("c")
        pltpu.sync_copy(x_hbm, xtab)

        def chunk_body(ci, _):
            base = wid * ept + ci * chunk
            pltpu.sync_copy(col_hbm.at[pl.ds(base, chunk)], idxc)
            pltpu.sync_copy(row_hbm.at[pl.ds(base, chunk)], idxr)
            ca = pltpu.async_copy(a_hbm.at[idxc], abuf, sem1)
            cb = pltpu.async_copy(b_hbm.at[idxr], bbuf, sem2)
            ca.wait()
            cb.wait()

            def add_body(i, _):
                for j in range(d // _L):
                    s = pl.ds(j * _L, _L)
                    pbuf[i, s] = abuf[i, s] + bbuf[i, s]
                return 0
            lax.fori_loop(0, chunk, add_body, 0)

            for v in range(chunk // _L):
                s = pl.ds(v * _L, _L)
                cv = idxc[s] * 4
                rv = idxr[s] * 4
                dxc = plsc.load_gather(xtab, [cv]) - plsc.load_gather(xtab, [rv])
                dyc = (plsc.load_gather(xtab, [cv + 1])
                       - plsc.load_gather(xtab, [rv + 1]))
                dzc = (plsc.load_gather(xtab, [cv + 2])
                       - plsc.load_gather(xtab, [rv + 2]))
                r2v = dxc * dxc + dyc * dyc + dzc * dzc
                rxb[s] = dxc
                ryb[s] = dyc
                rzb[s] = dzc
                r2b[s] = r2v
                potb[s] = 1.0 / (r2v + 1e-6)

            pltpu.sync_copy(pbuf, p_hbm.at[pl.ds(base, chunk)])
            pltpu.sync_copy(r2b, r2_hbm.at[pl.ds(base, chunk)])
            pltpu.sync_copy(potb, pot_hbm.at[pl.ds(base, chunk)])
            pltpu.sync_copy(rxb, rx_hbm.at[pl.ds(base, chunk)])
            pltpu.sync_copy(ryb, ry_hbm.at[pl.ds(base, chunk)])
            pltpu.sync_copy(rzb, rz_hbm.at[pl.ds(base, chunk)])
            return 0

        lax.fori_loop(0, nchunks, chunk_body, 0)

    return k(A, B, x4flat, col, row)


def _k3_body(p_ref, r2_ref, pot_ref, rx_ref, ry_ref, rz_ref,
             w1c_ref, w1d_ref, b1_ref, g1_ref, be1_ref,
             w2_ref, b2_ref, g2_ref, be2_ref, wg_ref,
             w5_ref, b5_ref, g5_ref, be5_ref, w6_ref, b6_ref,
             mij_ref, dx_ref):
    r2 = r2_ref[...]
    pot = pot_ref[...]
    pre1 = p_ref[...] + r2 * w1c_ref[...] + pot * w1d_ref[...] + b1_ref[...]
    u = _silu(_ln(pre1, g1_ref[...], be1_ref[...]))
    m = jnp.dot(u, w2_ref[...], preferred_element_type=jnp.float32) + b2_ref[...]
    m = _silu(_ln(m, g2_ref[...], be2_ref[...]))
    gate = jax.nn.sigmoid(jnp.dot(m, wg_ref[...], preferred_element_type=jnp.float32))
    mij = m * gate
    mij_ref[...] = mij
    t = jnp.dot(mij, w5_ref[...], preferred_element_type=jnp.float32) + b5_ref[...]
    t = _silu(_ln(t, g5_ref[...], be5_ref[...]))
    w = jax.nn.sigmoid(jnp.dot(t, w6_ref[...], preferred_element_type=jnp.float32)
                       + b6_ref[...])
    zero = jnp.zeros_like(w)
    dx_ref[...] = jnp.concatenate(
        [rx_ref[...] * w, ry_ref[...] * w, rz_ref[...] * w, zero], axis=-1)


def _edge_mlp(p, r2, pot, rx, ry, rz,
              w1c, w1d, b1, g1, be1, w2, b2, g2, be2, wg,
              w5, b5, g5, be5, w6, b6, be_blk=2000):
    e, d = p.shape
    grid = (e // be_blk,)
    col1 = lambda i: (i, 0)
    full = lambda i: (0, 0)
    spec_e1 = pl.BlockSpec((be_blk, 1), col1)
    spec_dd = pl.BlockSpec((d, d), full)
    spec_1d = pl.BlockSpec((1, d), full)
    spec_d1 = pl.BlockSpec((d, 1), full)
    spec_11 = pl.BlockSpec((1, 1), full)
    return pl.pallas_call(
        _k3_body,
        grid=grid,
        in_specs=[
            pl.BlockSpec((be_blk, d), col1),
            spec_e1, spec_e1, spec_e1, spec_e1, spec_e1,
            spec_1d, spec_1d, spec_1d, spec_1d, spec_1d,
            spec_dd, spec_1d, spec_1d, spec_1d, spec_d1,
            spec_dd, spec_1d, spec_1d, spec_1d, spec_d1, spec_11,
        ],
        out_specs=[
            pl.BlockSpec((be_blk, d), col1),
            pl.BlockSpec((be_blk, 4), col1),
        ],
        out_shape=[
            jax.ShapeDtypeStruct((e, d), jnp.float32),
            jax.ShapeDtypeStruct((e, 4), jnp.float32),
        ],
    )(p, r2, pot, rx, ry, rz, w1c, w1d, b1, g1, be1,
      w2, b2, g2, be2, wg, w5, b5, g5, be5, w6, b6)


def _sc_scatter(mij, dx4, col, row, npad, chunk=80):
    e, d = mij.shape
    ept = e // _NW
    nchunks = ept // chunk
    npt = npad // _NS
    mesh = plsc.VectorSubcoreMesh(core_axis_name="c", subcore_axis_name="s")

    @functools.partial(
        pl.kernel, mesh=mesh,
        out_type=(
            jax.ShapeDtypeStruct((_NC, npad, d), jnp.float32),
            jax.ShapeDtypeStruct((_NC, npad, d), jnp.float32),
        ),
        compiler_params=pltpu.CompilerParams(needs_layout_passes=False),
        scratch_types=[
            pltpu.VMEM((chunk,), jnp.int32),
            pltpu.VMEM((chunk, d), jnp.float32),
            pltpu.VMEM((chunk * 4,), jnp.float32),
            pltpu.VMEM((chunk, d), jnp.float32),
            pltpu.VMEM_SHARED((npad, d), jnp.float32),
        ],
    )
    def k(m_hbm, dx_hbm, col_hbm, row_hbm, zm_hbm, om_hbm, ox_hbm,
          idxc, mbuf, dxbuf, xpand, sm):
        cid = lax.axis_index("c")
        sid = lax.axis_index("s")
        wid = sid * _NC + cid
        nbase = sid * npt

        pltpu.sync_copy(zm_hbm.at[pl.ds(nbase, npt)], sm.at[pl.ds(nbase, npt)])
        zv = jnp.zeros((_L,), jnp.float32)

        def zexp_body(i, _):
            for j in range(d // _L):
                xpand[i, pl.ds(j * _L, _L)] = zv
            return 0
        lax.fori_loop(0, chunk, zexp_body, 0)
        plsc.subcore_barrier()

        def m_body(ci, _):
            base = wid * ept + ci * chunk
            pltpu.sync_copy(col_hbm.at[pl.ds(base, chunk)], idxc)
            pltpu.sync_copy(m_hbm.at[pl.ds(base, chunk)], mbuf)
            pltpu.sync_copy(mbuf, sm.at[idxc], add=True)
            return 0

        lax.fori_loop(0, nchunks, m_body, 0)
        plsc.subcore_barrier()
        pltpu.sync_copy(sm.at[pl.ds(nbase, npt)], om_hbm.at[cid, pl.ds(nbase, npt)])
        plsc.subcore_barrier()

        pltpu.sync_copy(zm_hbm.at[pl.ds(nbase, npt)], sm.at[pl.ds(nbase, npt)])
        plsc.subcore_barrier()

        lane = lax.iota(jnp.int32, _L)
        eoff = lax.shift_right_logical(lane, 2)
        coff = lax.bitwise_and(lane, 3)

        def dx_body(ci, _):
            base = wid * ept + ci * chunk
            pltpu.sync_copy(row_hbm.at[pl.ds(base, chunk)], idxc)
            pltpu.sync_copy(dx_hbm.at[pl.ds(base * 4, chunk * 4)], dxbuf)
            for v in range(chunk * 4 // _L):
                vals = dxbuf[pl.ds(v * _L, _L)]
                plsc.store_scatter(xpand, [4 * v + eoff, coff], vals)
            pltpu.sync_copy(xpand, sm.at[idxc], add=True)
            return 0

        lax.fori_loop(0, nchunks, dx_body, 0)
        plsc.subcore_barrier()
        pltpu.sync_copy(sm.at[pl.ds(nbase, npt)], ox_hbm.at[cid, pl.ds(nbase, npt)])

    return k(mij, dx4.reshape(-1), col, row, jnp.zeros((npad, d), jnp.float32))


def _k5_body(h_ref, mi0_ref, mi1_ref, x4_ref, dx0_ref, dx1_ref,
             w3a_ref, w3b_ref, b3_ref, g3_ref, be3_ref,
             w4_ref, b4_ref, g4_ref, be4_ref,
             hn_ref, xn_ref):
    h = h_ref[...]
    mi = mi0_ref[...] + mi1_ref[...]
    q = (jnp.dot(h, w3a_ref[...], preferred_element_type=jnp.float32)
         + jnp.dot(mi, w3b_ref[...], preferred_element_type=jnp.float32)
         + b3_ref[...])
    q = _silu(_ln(q, g3_ref[...], be3_ref[...]))
    ph = jnp.dot(q, w4_ref[...], preferred_element_type=jnp.float32) + b4_ref[...]
    ph = _ln(ph, g4_ref[...], be4_ref[...])
    hn_ref[...] = h + ph
    xn_ref[...] = x4_ref[...] + dx0_ref[...] + dx1_ref[...]


def _node_mlp(h, mi0, mi1, x4, dx0, dx1,
              w3a, w3b, b3, g3, be3, w4, b4, g4, be4, bn=1000):
    n, d = h.shape
    grid = (n // bn,)
    col1 = lambda i: (i, 0)
    full = lambda i: (0, 0)
    spec_dd = pl.BlockSpec((d, d), full)
    spec_1d = pl.BlockSpec((1, d), full)
    return pl.pallas_call(
        _k5_body,
        grid=grid,
        in_specs=[
            pl.BlockSpec((bn, d), col1),
            pl.BlockSpec((bn, d), col1),
            pl.BlockSpec((bn, d), col1),
            pl.BlockSpec((bn, 4), col1),
            pl.BlockSpec((bn, 4), col1),
            pl.BlockSpec((bn, 4), col1),
            spec_dd, spec_dd, spec_1d, spec_1d, spec_1d,
            spec_dd, spec_1d, spec_1d, spec_1d,
        ],
        out_specs=[
            pl.BlockSpec((bn, d), col1),
            pl.BlockSpec((bn, 4), col1),
        ],
        out_shape=[
            jax.ShapeDtypeStruct((n, d), jnp.float32),
            jax.ShapeDtypeStruct((n, 4), jnp.float32),
        ],
    )(h, mi0, mi1, x4, dx0, dx1, w3a, w3b, b3, g3, be3, w4, b4, g4, be4)


def kernel(x, h, edge_index, local_frames, batch,
           W1, b1, g1, be1, W2, b2, g2, be2, Wg,
           W3, b3, g3, be3, W4, b4, g4, be4,
           W5, b5, g5, be5, W6, b6):
    n, din = h.shape
    e = edge_index.shape[1]
    row = edge_index[0]
    col = edge_index[1]

    w1a = W1[:din]
    w1b = W1[din:2 * din]
    w1c = W1[2 * din:2 * din + 1]
    w1d = W1[2 * din + 1:2 * din + 2]

    A, B = _node_proj(h, w1a, w1b)

    x4 = jnp.concatenate([x, jnp.zeros((n, 1), jnp.float32)], axis=-1)
    p, r2, pot, rx, ry, rz = _sc_gather(A, B, x4.reshape(-1), col, row)
    r2 = r2.reshape(e, 1)
    pot = pot.reshape(e, 1)
    rx = rx.reshape(e, 1)
    ry = ry.reshape(e, 1)
    rz = rz.reshape(e, 1)

    mij, dx4 = _edge_mlp(
        p, r2, pot, rx, ry, rz,
        w1c, w1d, b1.reshape(1, -1), g1.reshape(1, -1), be1.reshape(1, -1),
        W2, b2.reshape(1, -1), g2.reshape(1, -1), be2.reshape(1, -1), Wg,
        W5, b5.reshape(1, -1), g5.reshape(1, -1), be5.reshape(1, -1),
        W6, b6.reshape(1, 1))

    npad = ((n + 8 * _NS - 1) // (8 * _NS)) * (8 * _NS)
    om, ox = _sc_scatter(mij, dx4, col, row, npad)

    hn, xn4 = _node_mlp(
        h, om[0, :n], om[1, :n], x4, ox[0, :n, :4], ox[1, :n, :4],
        W3[:din], W3[din:], b3.reshape(1, -1), g3.reshape(1, -1),
        be3.reshape(1, -1), W4, b4.reshape(1, -1), g4.reshape(1, -1),
        be4.reshape(1, -1))

    return (xn4[:, :3], hn)


# speedup vs baseline: 3.0585x; 1.2793x over previous
import functools

import jax
import jax.numpy as jnp
from jax import lax
from jax.experimental import pallas as pl
from jax.experimental.pallas import tpu as pltpu
from jax.experimental.pallas import tpu_sc as plsc

_EPS = 1e-5
_NC, _NS, _L = 2, 16, 16
_NW = _NC * _NS


def _ln(t, g, b):
    mu = jnp.mean(t, axis=-1, keepdims=True)
    var = jnp.mean((t - mu) ** 2, axis=-1, keepdims=True)
    return (t - mu) * lax.rsqrt(var + _EPS) * g + b


def _silu(t):
    return t * jax.nn.sigmoid(t)


def _k1_body(h_ref, wa_ref, wb_ref, a_ref, b_ref):
    h = h_ref[...]
    a_ref[...] = jnp.dot(h, wa_ref[...], preferred_element_type=jnp.float32)
    b_ref[...] = jnp.dot(h, wb_ref[...], preferred_element_type=jnp.float32)


def _node_proj(h, wa, wb, bn=1000):
    n, d = h.shape
    grid = (n // bn,)
    return pl.pallas_call(
        _k1_body,
        grid=grid,
        in_specs=[
            pl.BlockSpec((bn, d), lambda i: (i, 0)),
            pl.BlockSpec((d, d), lambda i: (0, 0)),
            pl.BlockSpec((d, d), lambda i: (0, 0)),
        ],
        out_specs=[
            pl.BlockSpec((bn, d), lambda i: (i, 0)),
            pl.BlockSpec((bn, d), lambda i: (i, 0)),
        ],
        out_shape=[
            jax.ShapeDtypeStruct((n, d), jnp.float32),
            jax.ShapeDtypeStruct((n, d), jnp.float32),
        ],
    )(h, wa, wb)


def _sc_gather(A, B, x4flat, col, row, chunk=80):
    n, d = A.shape
    e = col.shape[0]
    ept = e // _NW
    nchunks = ept // chunk
    npairs = (nchunks - 1) // 2
    assert nchunks == 2 * npairs + 1
    sb = 5 * chunk
    mesh = plsc.VectorSubcoreMesh(core_axis_name="c", subcore_axis_name="s")

    @functools.partial(
        pl.kernel, mesh=mesh,
        out_type=(
            jax.ShapeDtypeStruct((e, d), jnp.float32),
            jax.ShapeDtypeStruct((5 * e,), jnp.float32),
        ),
        compiler_params=pltpu.CompilerParams(needs_layout_passes=False),
        scratch_types=[
            pltpu.VMEM((chunk,), jnp.int32),
            pltpu.VMEM((chunk,), jnp.int32),
            pltpu.VMEM((chunk,), jnp.int32),
            pltpu.VMEM((chunk,), jnp.int32),
            pltpu.VMEM((chunk, d), jnp.float32),
            pltpu.VMEM((chunk, d), jnp.float32),
            pltpu.VMEM((chunk, d), jnp.float32),
            pltpu.VMEM((chunk, d), jnp.float32),
            pltpu.VMEM((chunk, d), jnp.float32),
            pltpu.VMEM((n * 4,), jnp.float32),
            pltpu.VMEM((sb,), jnp.float32),
            pltpu.SemaphoreType.DMA,
            pltpu.SemaphoreType.DMA,
        ],
    )
    def k(a_hbm, b_hbm, x_hbm, col_hbm, row_hbm,
          p_hbm, s_hbm,
          idxc0, idxr0, idxc1, idxr1, abuf0, bbuf0, abuf1, bbuf1,
          pbuf, xtab, sbuf, sem0, sem1):
        wid = lax.axis_index("s") * _NC + lax.axis_index("c")
        pltpu.sync_copy(x_hbm, xtab)

        def fire(ci, idxc, idxr, abuf, bbuf, sem):
            base = wid * ept + ci * chunk
            pltpu.sync_copy(col_hbm.at[pl.ds(base, chunk)], idxc)
            pltpu.sync_copy(row_hbm.at[pl.ds(base, chunk)], idxr)
            pltpu.async_copy(a_hbm.at[idxc], abuf, sem)
            pltpu.async_copy(b_hbm.at[idxr], bbuf, sem)

        def drain(idxc, idxr, abuf, bbuf, sem):
            pltpu.make_async_copy(a_hbm.at[idxc], abuf, sem).wait()
            pltpu.make_async_copy(b_hbm.at[idxr], bbuf, sem).wait()

        def compute_out(ci, idxc, idxr, abuf, bbuf):
            base = wid * ept + ci * chunk

            def add_body(i, _):
                for j in range(d // _L):
                    s = pl.ds(j * _L, _L)
                    pbuf[i, s] = abuf[i, s] + bbuf[i, s]
                return 0
            lax.fori_loop(0, chunk, add_body, 0)

            for v in range(chunk // _L):
                s = pl.ds(v * _L, _L)
                cv = idxc[s] * 4
                rv = idxr[s] * 4
                dxc = plsc.load_gather(xtab, [cv]) - plsc.load_gather(xtab, [rv])
                dyc = (plsc.load_gather(xtab, [cv + 1])
                       - plsc.load_gather(xtab, [rv + 1]))
                dzc = (plsc.load_gather(xtab, [cv + 2])
                       - plsc.load_gather(xtab, [rv + 2]))
                r2v = dxc * dxc + dyc * dyc + dzc * dzc
                sbuf[pl.ds(v * _L, _L)] = dxc
                sbuf[pl.ds(chunk + v * _L, _L)] = dyc
                sbuf[pl.ds(2 * chunk + v * _L, _L)] = dzc
                sbuf[pl.ds(3 * chunk + v * _L, _L)] = r2v
                sbuf[pl.ds(4 * chunk + v * _L, _L)] = 1.0 / (r2v + 1e-6)

            pltpu.sync_copy(pbuf, p_hbm.at[pl.ds(base, chunk)])
            soff = (wid * nchunks + ci) * sb
            pltpu.sync_copy(sbuf, s_hbm.at[pl.ds(soff, sb)])

        fire(0, idxc0, idxr0, abuf0, bbuf0, sem0)

        def pair_body(i, _):
            c0 = 2 * i
            fire(c0 + 1, idxc1, idxr1, abuf1, bbuf1, sem1)
            drain(idxc0, idxr0, abuf0, bbuf0, sem0)
            compute_out(c0, idxc0, idxr0, abuf0, bbuf0)
            fire(c0 + 2, idxc0, idxr0, abuf0, bbuf0, sem0)
            drain(idxc1, idxr1, abuf1, bbuf1, sem1)
            compute_out(c0 + 1, idxc1, idxr1, abuf1, bbuf1)
            return 0

        lax.fori_loop(0, npairs, pair_body, 0)
        drain(idxc0, idxr0, abuf0, bbuf0, sem0)
        compute_out(nchunks - 1, idxc0, idxr0, abuf0, bbuf0)

    return k(A, B, x4flat, col, row)


def _k3_body(p_ref, r2_ref, pot_ref, rx_ref, ry_ref, rz_ref,
             w1c_ref, w1d_ref, b1_ref, g1_ref, be1_ref,
             w2_ref, b2_ref, g2_ref, be2_ref, wg_ref,
             w5_ref, b5_ref, g5_ref, be5_ref, w6_ref, b6_ref,
             mij_ref, dx_ref):
    r2 = r2_ref[...]
    pot = pot_ref[...]
    pre1 = p_ref[...] + r2 * w1c_ref[...] + pot * w1d_ref[...] + b1_ref[...]
    u = _silu(_ln(pre1, g1_ref[...], be1_ref[...]))
    m = jnp.dot(u, w2_ref[...], preferred_element_type=jnp.float32) + b2_ref[...]
    m = _silu(_ln(m, g2_ref[...], be2_ref[...]))
    gate = jax.nn.sigmoid(jnp.dot(m, wg_ref[...], preferred_element_type=jnp.float32))
    mij = m * gate
    mij_ref[...] = mij
    t = jnp.dot(mij, w5_ref[...], preferred_element_type=jnp.float32) + b5_ref[...]
    t = _silu(_ln(t, g5_ref[...], be5_ref[...]))
    w = jax.nn.sigmoid(jnp.dot(t, w6_ref[...], preferred_element_type=jnp.float32)
                       + b6_ref[...])
    zero = jnp.zeros_like(w)
    dx_ref[...] = jnp.concatenate(
        [rx_ref[...] * w, ry_ref[...] * w, rz_ref[...] * w, zero], axis=-1)


def _edge_mlp(p, r2, pot, rx, ry, rz,
              w1c, w1d, b1, g1, be1, w2, b2, g2, be2, wg,
              w5, b5, g5, be5, w6, b6, be_blk=2000):
    e, d = p.shape
    grid = (e // be_blk,)
    col1 = lambda i: (i, 0)
    full = lambda i: (0, 0)
    spec_e1 = pl.BlockSpec((be_blk, 1), col1)
    spec_dd = pl.BlockSpec((d, d), full)
    spec_1d = pl.BlockSpec((1, d), full)
    spec_d1 = pl.BlockSpec((d, 1), full)
    spec_11 = pl.BlockSpec((1, 1), full)
    return pl.pallas_call(
        _k3_body,
        grid=grid,
        in_specs=[
            pl.BlockSpec((be_blk, d), col1),
            spec_e1, spec_e1, spec_e1, spec_e1, spec_e1,
            spec_1d, spec_1d, spec_1d, spec_1d, spec_1d,
            spec_dd, spec_1d, spec_1d, spec_1d, spec_d1,
            spec_dd, spec_1d, spec_1d, spec_1d, spec_d1, spec_11,
        ],
        out_specs=[
            pl.BlockSpec((be_blk, d), col1),
            pl.BlockSpec((be_blk, 4), col1),
        ],
        out_shape=[
            jax.ShapeDtypeStruct((e, d), jnp.float32),
            jax.ShapeDtypeStruct((e, 4), jnp.float32),
        ],
    )(p, r2, pot, rx, ry, rz, w1c, w1d, b1, g1, be1,
      w2, b2, g2, be2, wg, w5, b5, g5, be5, w6, b6)


def _sc_scatter(mij, dx4, col, row, npad, chunk=80):
    e, d = mij.shape
    ept = e // _NW
    nchunks = ept // chunk
    npt = npad // _NS
    mesh = plsc.VectorSubcoreMesh(core_axis_name="c", subcore_axis_name="s")

    npairs = (nchunks - 1) // 2
    assert nchunks == 2 * npairs + 1

    @functools.partial(
        pl.kernel, mesh=mesh,
        out_type=(
            jax.ShapeDtypeStruct((_NC, npad, d), jnp.float32),
            jax.ShapeDtypeStruct((_NC, npad, d), jnp.float32),
        ),
        compiler_params=pltpu.CompilerParams(needs_layout_passes=False),
        scratch_types=[
            pltpu.VMEM((chunk,), jnp.int32),
            pltpu.VMEM((chunk,), jnp.int32),
            pltpu.VMEM((chunk, d), jnp.float32),
            pltpu.VMEM((chunk, d), jnp.float32),
            pltpu.VMEM((chunk * 4,), jnp.float32),
            pltpu.VMEM((chunk * 4,), jnp.float32),
            pltpu.VMEM((chunk, d), jnp.float32),
            pltpu.VMEM_SHARED((npad, d), jnp.float32),
            pltpu.SemaphoreType.DMA,
            pltpu.SemaphoreType.DMA,
        ],
    )
    def k(m_hbm, dx_hbm, col_hbm, row_hbm, zm_hbm, om_hbm, ox_hbm,
          idx0, idx1, mbuf0, mbuf1, dxbuf0, dxbuf1, xpand, sm, sem0, sem1):
        cid = lax.axis_index("c")
        sid = lax.axis_index("s")
        wid = sid * _NC + cid
        nbase = sid * npt

        pltpu.sync_copy(zm_hbm.at[pl.ds(nbase, npt)], sm.at[pl.ds(nbase, npt)])
        zv = jnp.zeros((_L,), jnp.float32)

        def zexp_body(i, _):
            for j in range(d // _L):
                xpand[i, pl.ds(j * _L, _L)] = zv
            return 0
        lax.fori_loop(0, chunk, zexp_body, 0)
        plsc.subcore_barrier()

        def m_fire(ci, idx, mbuf, sem):
            base = wid * ept + ci * chunk
            pltpu.sync_copy(col_hbm.at[pl.ds(base, chunk)], idx)
            pltpu.async_copy(m_hbm.at[pl.ds(base, chunk)], mbuf, sem)

        def m_scat(ci, idx, mbuf, sem):
            base = wid * ept + ci * chunk
            pltpu.make_async_copy(
                m_hbm.at[pl.ds(base, chunk)], mbuf, sem).wait()
            pltpu.sync_copy(mbuf, sm.at[idx], add=True)

        m_fire(0, idx0, mbuf0, sem0)

        def m_pair(i, _):
            c0 = 2 * i
            m_fire(c0 + 1, idx1, mbuf1, sem1)
            m_scat(c0, idx0, mbuf0, sem0)
            m_fire(c0 + 2, idx0, mbuf0, sem0)
            m_scat(c0 + 1, idx1, mbuf1, sem1)
            return 0

        lax.fori_loop(0, npairs, m_pair, 0)
        m_scat(nchunks - 1, idx0, mbuf0, sem0)

        plsc.subcore_barrier()
        pltpu.sync_copy(sm.at[pl.ds(nbase, npt)], om_hbm.at[cid, pl.ds(nbase, npt)])
        plsc.subcore_barrier()

        pltpu.sync_copy(zm_hbm.at[pl.ds(nbase, npt)], sm.at[pl.ds(nbase, npt)])
        plsc.subcore_barrier()

        lane = lax.iota(jnp.int32, _L)
        eoff = lax.shift_right_logical(lane, 2)
        coff = lax.bitwise_and(lane, 3)

        def dx_fire(ci, idx, dxbuf, sem):
            base = wid * ept + ci * chunk
            pltpu.sync_copy(row_hbm.at[pl.ds(base, chunk)], idx)
            pltpu.async_copy(dx_hbm.at[pl.ds(base * 4, chunk * 4)], dxbuf, sem)

        def dx_scat(ci, idx, dxbuf, sem):
            base = wid * ept + ci * chunk
            pltpu.make_async_copy(
                dx_hbm.at[pl.ds(base * 4, chunk * 4)], dxbuf, sem).wait()
            for v in range(chunk * 4 // _L):
                vals = dxbuf[pl.ds(v * _L, _L)]
                plsc.store_scatter(xpand, [4 * v + eoff, coff], vals)
            pltpu.sync_copy(xpand, sm.at[idx], add=True)

        dx_fire(0, idx0, dxbuf0, sem0)

        def dx_pair(i, _):
            c0 = 2 * i
            dx_fire(c0 + 1, idx1, dxbuf1, sem1)
            dx_scat(c0, idx0, dxbuf0, sem0)
            dx_fire(c0 + 2, idx0, dxbuf0, sem0)
            dx_scat(c0 + 1, idx1, dxbuf1, sem1)
            return 0

        lax.fori_loop(0, npairs, dx_pair, 0)
        dx_scat(nchunks - 1, idx0, dxbuf0, sem0)

        plsc.subcore_barrier()
        pltpu.sync_copy(sm.at[pl.ds(nbase, npt)], ox_hbm.at[cid, pl.ds(nbase, npt)])

    return k(mij, dx4.reshape(-1), col, row, jnp.zeros((npad, d), jnp.float32))


def _k5_body(h_ref, mi0_ref, mi1_ref, x4_ref, dx0_ref, dx1_ref,
             w3a_ref, w3b_ref, b3_ref, g3_ref, be3_ref,
             w4_ref, b4_ref, g4_ref, be4_ref,
             hn_ref, xn_ref):
    h = h_ref[...]
    mi = mi0_ref[...] + mi1_ref[...]
    q = (jnp.dot(h, w3a_ref[...], preferred_element_type=jnp.float32)
         + jnp.dot(mi, w3b_ref[...], preferred_element_type=jnp.float32)
         + b3_ref[...])
    q = _silu(_ln(q, g3_ref[...], be3_ref[...]))
    ph = jnp.dot(q, w4_ref[...], preferred_element_type=jnp.float32) + b4_ref[...]
    ph = _ln(ph, g4_ref[...], be4_ref[...])
    hn_ref[...] = h + ph
    xn_ref[...] = x4_ref[...] + dx0_ref[...] + dx1_ref[...]


def _node_mlp(h, mi0, mi1, x4, dx0, dx1,
              w3a, w3b, b3, g3, be3, w4, b4, g4, be4, bn=1000):
    n, d = h.shape
    grid = (n // bn,)
    col1 = lambda i: (i, 0)
    full = lambda i: (0, 0)
    spec_dd = pl.BlockSpec((d, d), full)
    spec_1d = pl.BlockSpec((1, d), full)
    return pl.pallas_call(
        _k5_body,
        grid=grid,
        in_specs=[
            pl.BlockSpec((bn, d), col1),
            pl.BlockSpec((bn, d), col1),
            pl.BlockSpec((bn, d), col1),
            pl.BlockSpec((bn, 4), col1),
            pl.BlockSpec((bn, 4), col1),
            pl.BlockSpec((bn, 4), col1),
            spec_dd, spec_dd, spec_1d, spec_1d, spec_1d,
            spec_dd, spec_1d, spec_1d, spec_1d,
        ],
        out_specs=[
            pl.BlockSpec((bn, d), col1),
            pl.BlockSpec((bn, 4), col1),
        ],
        out_shape=[
            jax.ShapeDtypeStruct((n, d), jnp.float32),
            jax.ShapeDtypeStruct((n, 4), jnp.float32),
        ],
    )(h, mi0, mi1, x4, dx0, dx1, w3a, w3b, b3, g3, be3, w4, b4, g4, be4)


def kernel(x, h, edge_index, local_frames, batch,
           W1, b1, g1, be1, W2, b2, g2, be2, Wg,
           W3, b3, g3, be3, W4, b4, g4, be4,
           W5, b5, g5, be5, W6, b6):
    n, din = h.shape
    e = edge_index.shape[1]
    row = edge_index[0]
    col = edge_index[1]

    w1a = W1[:din]
    w1b = W1[din:2 * din]
    w1c = W1[2 * din:2 * din + 1]
    w1d = W1[2 * din + 1:2 * din + 2]

    A, B = _node_proj(h, w1a, w1b)

    x4 = jnp.concatenate([x, jnp.zeros((n, 1), jnp.float32)], axis=-1)
    p, s = _sc_gather(A, B, x4.reshape(-1), col, row)
    chunk = 80
    nch = e // _NW // chunk
    S = s.reshape(_NW, nch, 5, chunk).transpose(2, 0, 1, 3).reshape(5, e)
    rx = S[0].reshape(e, 1)
    ry = S[1].reshape(e, 1)
    rz = S[2].reshape(e, 1)
    r2 = S[3].reshape(e, 1)
    pot = S[4].reshape(e, 1)

    mij, dx4 = _edge_mlp(
        p, r2, pot, rx, ry, rz,
        w1c, w1d, b1.reshape(1, -1), g1.reshape(1, -1), be1.reshape(1, -1),
        W2, b2.reshape(1, -1), g2.reshape(1, -1), be2.reshape(1, -1), Wg,
        W5, b5.reshape(1, -1), g5.reshape(1, -1), be5.reshape(1, -1),
        W6, b6.reshape(1, 1))

    npad = ((n + 8 * _NS - 1) // (8 * _NS)) * (8 * _NS)
    om, ox = _sc_scatter(mij, dx4, col, row, npad)

    hn, xn4 = _node_mlp(
        h, om[0, :n], om[1, :n], x4, ox[0, :n, :4], ox[1, :n, :4],
        W3[:din], W3[din:], b3.reshape(1, -1), g3.reshape(1, -1),
        be3.reshape(1, -1), W4, b4.reshape(1, -1), g4.reshape(1, -1),
        be4.reshape(1, -1))

    return (xn4[:, :3], hn)


# speedup vs baseline: 3.1401x; 1.0267x over previous
import functools

import jax
import jax.numpy as jnp
from jax import lax
from jax.experimental import pallas as pl
from jax.experimental.pallas import tpu as pltpu
from jax.experimental.pallas import tpu_sc as plsc

_EPS = 1e-5
_NC, _NS, _L = 2, 16, 16
_NW = _NC * _NS


def _ln(t, g, b):
    mu = jnp.mean(t, axis=-1, keepdims=True)
    var = jnp.mean((t - mu) ** 2, axis=-1, keepdims=True)
    return (t - mu) * lax.rsqrt(var + _EPS) * g + b


def _silu(t):
    return t * jax.nn.sigmoid(t)


def _k1_body(h_ref, wa_ref, wb_ref, a_ref, b_ref):
    h = h_ref[...]
    a_ref[...] = jnp.dot(h, wa_ref[...], preferred_element_type=jnp.float32)
    b_ref[...] = jnp.dot(h, wb_ref[...], preferred_element_type=jnp.float32)


def _node_proj(h, wa, wb, bn=1000):
    n, d = h.shape
    grid = (n // bn,)
    return pl.pallas_call(
        _k1_body,
        grid=grid,
        in_specs=[
            pl.BlockSpec((bn, d), lambda i: (i, 0)),
            pl.BlockSpec((d, d), lambda i: (0, 0)),
            pl.BlockSpec((d, d), lambda i: (0, 0)),
        ],
        out_specs=[
            pl.BlockSpec((bn, d), lambda i: (i, 0)),
            pl.BlockSpec((bn, d), lambda i: (i, 0)),
        ],
        out_shape=[
            jax.ShapeDtypeStruct((n, d), jnp.float32),
            jax.ShapeDtypeStruct((n, d), jnp.float32),
        ],
    )(h, wa, wb)


def _sc_gather(A, B, x4flat, col, row, chunk=80):
    n, d = A.shape
    e = col.shape[0]
    ept = e // _NW
    nchunks = ept // chunk
    npairs = (nchunks - 1) // 2
    assert nchunks == 2 * npairs + 1
    sb = 5 * chunk
    mesh = plsc.VectorSubcoreMesh(core_axis_name="c", subcore_axis_name="s")

    @functools.partial(
        pl.kernel, mesh=mesh,
        out_type=(
            jax.ShapeDtypeStruct((e, d), jnp.float32),
            jax.ShapeDtypeStruct((5 * e,), jnp.float32),
        ),
        compiler_params=pltpu.CompilerParams(needs_layout_passes=False),
        scratch_types=[
            pltpu.VMEM((chunk,), jnp.int32),
            pltpu.VMEM((chunk,), jnp.int32),
            pltpu.VMEM((chunk,), jnp.int32),
            pltpu.VMEM((chunk,), jnp.int32),
            pltpu.VMEM((chunk, d), jnp.float32),
            pltpu.VMEM((chunk, d), jnp.float32),
            pltpu.VMEM((chunk, d), jnp.float32),
            pltpu.VMEM((chunk, d), jnp.float32),
            pltpu.VMEM((chunk, d), jnp.float32),
            pltpu.VMEM((chunk, d), jnp.float32),
            pltpu.VMEM((n * 4,), jnp.float32),
            pltpu.VMEM((sb,), jnp.float32),
            pltpu.VMEM((sb,), jnp.float32),
            pltpu.SemaphoreType.DMA,
            pltpu.SemaphoreType.DMA,
            pltpu.SemaphoreType.DMA,
            pltpu.SemaphoreType.DMA,
        ],
    )
    def k(a_hbm, b_hbm, x_hbm, col_hbm, row_hbm,
          p_hbm, s_hbm,
          idxc0, idxr0, idxc1, idxr1, abuf0, bbuf0, abuf1, bbuf1,
          pbuf0, pbuf1, xtab, sbuf0, sbuf1, sem0, sem1, osem0, osem1):
        wid = lax.axis_index("s") * _NC + lax.axis_index("c")
        pltpu.sync_copy(x_hbm, xtab)

        def fire(ci, idxc, idxr, abuf, bbuf, sem):
            base = wid * ept + ci * chunk
            pltpu.sync_copy(col_hbm.at[pl.ds(base, chunk)], idxc)
            pltpu.sync_copy(row_hbm.at[pl.ds(base, chunk)], idxr)
            pltpu.async_copy(a_hbm.at[idxc], abuf, sem)
            pltpu.async_copy(b_hbm.at[idxr], bbuf, sem)

        def drain(idxc, idxr, abuf, bbuf, sem):
            pltpu.make_async_copy(a_hbm.at[idxc], abuf, sem).wait()
            pltpu.make_async_copy(b_hbm.at[idxr], bbuf, sem).wait()

        def compute_out(ci, idxc, idxr, abuf, bbuf, pbuf, sbuf, osem,
                        drain_out):
            base = wid * ept + ci * chunk
            soff = (wid * nchunks + ci) * sb
            if drain_out:
                pltpu.make_async_copy(pbuf, p_hbm.at[pl.ds(base, chunk)],
                                      osem).wait()
                pltpu.make_async_copy(sbuf, s_hbm.at[pl.ds(soff, sb)],
                                      osem).wait()

            @plsc.parallel_loop(0, chunk, unroll=2)
            def add_body(i):
                for j in range(d // _L):
                    s = pl.ds(j * _L, _L)
                    pbuf[i, s] = abuf[i, s] + bbuf[i, s]

            for v in range(chunk // _L):
                s = pl.ds(v * _L, _L)
                cv = idxc[s] * 4
                rv = idxr[s] * 4
                dxc = plsc.load_gather(xtab, [cv]) - plsc.load_gather(xtab, [rv])
                dyc = (plsc.load_gather(xtab, [cv + 1])
                       - plsc.load_gather(xtab, [rv + 1]))
                dzc = (plsc.load_gather(xtab, [cv + 2])
                       - plsc.load_gather(xtab, [rv + 2]))
                r2v = dxc * dxc + dyc * dyc + dzc * dzc
                sbuf[pl.ds(v * _L, _L)] = dxc
                sbuf[pl.ds(chunk + v * _L, _L)] = dyc
                sbuf[pl.ds(2 * chunk + v * _L, _L)] = dzc
                sbuf[pl.ds(3 * chunk + v * _L, _L)] = r2v
                sbuf[pl.ds(4 * chunk + v * _L, _L)] = 1.0 / (r2v + 1e-6)

            pltpu.async_copy(pbuf, p_hbm.at[pl.ds(base, chunk)], osem)
            pltpu.async_copy(sbuf, s_hbm.at[pl.ds(soff, sb)], osem)

        fire(0, idxc0, idxr0, abuf0, bbuf0, sem0)
        fire(1, idxc1, idxr1, abuf1, bbuf1, sem1)
        drain(idxc0, idxr0, abuf0, bbuf0, sem0)
        compute_out(0, idxc0, idxr0, abuf0, bbuf0, pbuf0, sbuf0, osem0, False)
        fire(2, idxc0, idxr0, abuf0, bbuf0, sem0)
        drain(idxc1, idxr1, abuf1, bbuf1, sem1)
        compute_out(1, idxc1, idxr1, abuf1, bbuf1, pbuf1, sbuf1, osem1, False)

        def pair_body(i, _):
            c0 = 2 * i + 2
            fire(c0 + 1, idxc1, idxr1, abuf1, bbuf1, sem1)
            drain(idxc0, idxr0, abuf0, bbuf0, sem0)
            compute_out(c0, idxc0, idxr0, abuf0, bbuf0, pbuf0, sbuf0, osem0,
                        True)
            fire(c0 + 2, idxc0, idxr0, abuf0, bbuf0, sem0)
            drain(idxc1, idxr1, abuf1, bbuf1, sem1)
            compute_out(c0 + 1, idxc1, idxr1, abuf1, bbuf1, pbuf1, sbuf1,
                        osem1, True)
            return 0

        lax.fori_loop(0, npairs - 1, pair_body, 0)
        drain(idxc0, idxr0, abuf0, bbuf0, sem0)
        compute_out(nchunks - 1, idxc0, idxr0, abuf0, bbuf0, pbuf0, sbuf0,
                    osem0, True)
        base_l = wid * ept + (nchunks - 1) * chunk
        soff_l = (wid * nchunks + (nchunks - 1)) * sb
        pltpu.make_async_copy(pbuf0, p_hbm.at[pl.ds(base_l, chunk)],
                              osem0).wait()
        pltpu.make_async_copy(sbuf0, s_hbm.at[pl.ds(soff_l, sb)], osem0).wait()
        pltpu.make_async_copy(pbuf1, p_hbm.at[pl.ds(base_l, chunk)],
                              osem1).wait()
        pltpu.make_async_copy(sbuf1, s_hbm.at[pl.ds(soff_l, sb)], osem1).wait()

    return k(A, B, x4flat, col, row)


def _k3_body(p_ref, r2_ref, pot_ref, rx_ref, ry_ref, rz_ref,
             w1c_ref, w1d_ref, b1_ref, g1_ref, be1_ref,
             w2_ref, b2_ref, g2_ref, be2_ref, wg_ref,
             w5_ref, b5_ref, g5_ref, be5_ref, w6_ref, b6_ref,
             mij_ref, dx_ref):
    r2 = r2_ref[...]
    pot = pot_ref[...]
    pre1 = p_ref[...] + r2 * w1c_ref[...] + pot * w1d_ref[...] + b1_ref[...]
    u = _silu(_ln(pre1, g1_ref[...], be1_ref[...]))
    m = jnp.dot(u, w2_ref[...], preferred_element_type=jnp.float32) + b2_ref[...]
    m = _silu(_ln(m, g2_ref[...], be2_ref[...]))
    gate = jax.nn.sigmoid(jnp.dot(m, wg_ref[...], preferred_element_type=jnp.float32))
    mij = m * gate
    mij_ref[...] = mij
    t = jnp.dot(mij, w5_ref[...], preferred_element_type=jnp.float32) + b5_ref[...]
    t = _silu(_ln(t, g5_ref[...], be5_ref[...]))
    w = jax.nn.sigmoid(jnp.dot(t, w6_ref[...], preferred_element_type=jnp.float32)
                       + b6_ref[...])
    zero = jnp.zeros_like(w)
    dx_ref[...] = jnp.concatenate(
        [rx_ref[...] * w, ry_ref[...] * w, rz_ref[...] * w, zero], axis=-1)


def _edge_mlp(p, r2, pot, rx, ry, rz,
              w1c, w1d, b1, g1, be1, w2, b2, g2, be2, wg,
              w5, b5, g5, be5, w6, b6, be_blk=2000):
    e, d = p.shape
    grid = (e // be_blk,)
    col1 = lambda i: (i, 0)
    full = lambda i: (0, 0)
    spec_e1 = pl.BlockSpec((be_blk, 1), col1)
    spec_dd = pl.BlockSpec((d, d), full)
    spec_1d = pl.BlockSpec((1, d), full)
    spec_d1 = pl.BlockSpec((d, 1), full)
    spec_11 = pl.BlockSpec((1, 1), full)
    return pl.pallas_call(
        _k3_body,
        grid=grid,
        in_specs=[
            pl.BlockSpec((be_blk, d), col1),
            spec_e1, spec_e1, spec_e1, spec_e1, spec_e1,
            spec_1d, spec_1d, spec_1d, spec_1d, spec_1d,
            spec_dd, spec_1d, spec_1d, spec_1d, spec_d1,
            spec_dd, spec_1d, spec_1d, spec_1d, spec_d1, spec_11,
        ],
        out_specs=[
            pl.BlockSpec((be_blk, d), col1),
            pl.BlockSpec((be_blk, 4), col1),
        ],
        out_shape=[
            jax.ShapeDtypeStruct((e, d), jnp.float32),
            jax.ShapeDtypeStruct((e, 4), jnp.float32),
        ],
    )(p, r2, pot, rx, ry, rz, w1c, w1d, b1, g1, be1,
      w2, b2, g2, be2, wg, w5, b5, g5, be5, w6, b6)


def _sc_scatter(mij, dx4, col, row, npad, chunk=80):
    e, d = mij.shape
    ept = e // _NW
    nchunks = ept // chunk
    npt = npad // _NS
    mesh = plsc.VectorSubcoreMesh(core_axis_name="c", subcore_axis_name="s")

    npairs = (nchunks - 1) // 2
    assert nchunks == 2 * npairs + 1

    @functools.partial(
        pl.kernel, mesh=mesh,
        out_type=(
            jax.ShapeDtypeStruct((_NC, npad, d), jnp.float32),
            jax.ShapeDtypeStruct((_NC, npad, d), jnp.float32),
        ),
        compiler_params=pltpu.CompilerParams(needs_layout_passes=False),
        scratch_types=[
            pltpu.VMEM((chunk,), jnp.int32),
            pltpu.VMEM((chunk,), jnp.int32),
            pltpu.VMEM((chunk, d), jnp.float32),
            pltpu.VMEM((chunk, d), jnp.float32),
            pltpu.VMEM((chunk * 4,), jnp.float32),
            pltpu.VMEM((chunk * 4,), jnp.float32),
            pltpu.VMEM((chunk, d), jnp.float32),
            pltpu.VMEM_SHARED((npad, d), jnp.float32),
            pltpu.SemaphoreType.DMA,
            pltpu.SemaphoreType.DMA,
        ],
    )
    def k(m_hbm, dx_hbm, col_hbm, row_hbm, zm_hbm, om_hbm, ox_hbm,
          idx0, idx1, mbuf0, mbuf1, dxbuf0, dxbuf1, xpand, sm, sem0, sem1):
        cid = lax.axis_index("c")
        sid = lax.axis_index("s")
        wid = sid * _NC + cid
        nbase = sid * npt

        pltpu.sync_copy(zm_hbm.at[pl.ds(nbase, npt)], sm.at[pl.ds(nbase, npt)])
        zv = jnp.zeros((_L,), jnp.float32)

        def zexp_body(i, _):
            for j in range(d // _L):
                xpand[i, pl.ds(j * _L, _L)] = zv
            return 0
        lax.fori_loop(0, chunk, zexp_body, 0)
        plsc.subcore_barrier()

        def m_fire(ci, idx, mbuf, sem):
            base = wid * ept + ci * chunk
            pltpu.sync_copy(col_hbm.at[pl.ds(base, chunk)], idx)
            pltpu.async_copy(m_hbm.at[pl.ds(base, chunk)], mbuf, sem)

        def m_scat(ci, idx, mbuf, sem):
            base = wid * ept + ci * chunk
            pltpu.make_async_copy(
                m_hbm.at[pl.ds(base, chunk)], mbuf, sem).wait()
            pltpu.sync_copy(mbuf, sm.at[idx], add=True)

        m_fire(0, idx0, mbuf0, sem0)

        def m_pair(i, _):
            c0 = 2 * i
            m_fire(c0 + 1, idx1, mbuf1, sem1)
            m_scat(c0, idx0, mbuf0, sem0)
            m_fire(c0 + 2, idx0, mbuf0, sem0)
            m_scat(c0 + 1, idx1, mbuf1, sem1)
            return 0

        lax.fori_loop(0, npairs, m_pair, 0)
        m_scat(nchunks - 1, idx0, mbuf0, sem0)

        plsc.subcore_barrier()
        pltpu.sync_copy(sm.at[pl.ds(nbase, npt)], om_hbm.at[cid, pl.ds(nbase, npt)])
        plsc.subcore_barrier()

        pltpu.sync_copy(zm_hbm.at[pl.ds(nbase, npt)], sm.at[pl.ds(nbase, npt)])
        plsc.subcore_barrier()

        lane = lax.iota(jnp.int32, _L)
        eoff = lax.shift_right_logical(lane, 2)
        coff = lax.bitwise_and(lane, 3)

        def dx_fire(ci, idx, dxbuf, sem):
            base = wid * ept + ci * chunk
            pltpu.sync_copy(row_hbm.at[pl.ds(base, chunk)], idx)
            pltpu.async_copy(dx_hbm.at[pl.ds(base * 4, chunk * 4)], dxbuf, sem)

        def dx_scat(ci, idx, dxbuf, sem):
            base = wid * ept + ci * chunk
            pltpu.make_async_copy(
                dx_hbm.at[pl.ds(base * 4, chunk * 4)], dxbuf, sem).wait()
            for v in range(chunk * 4 // _L):
                vals = dxbuf[pl.ds(v * _L, _L)]
                plsc.store_scatter(xpand, [4 * v + eoff, coff], vals)
            pltpu.sync_copy(xpand, sm.at[idx], add=True)

        dx_fire(0, idx0, dxbuf0, sem0)

        def dx_pair(i, _):
            c0 = 2 * i
            dx_fire(c0 + 1, idx1, dxbuf1, sem1)
            dx_scat(c0, idx0, dxbuf0, sem0)
            dx_fire(c0 + 2, idx0, dxbuf0, sem0)
            dx_scat(c0 + 1, idx1, dxbuf1, sem1)
            return 0

        lax.fori_loop(0, npairs, dx_pair, 0)
        dx_scat(nchunks - 1, idx0, dxbuf0, sem0)

        plsc.subcore_barrier()
        pltpu.sync_copy(sm.at[pl.ds(nbase, npt)], ox_hbm.at[cid, pl.ds(nbase, npt)])

    return k(mij, dx4.reshape(-1), col, row, jnp.zeros((npad, d), jnp.float32))


def _k5_body(h_ref, mi0_ref, mi1_ref, x4_ref, dx0_ref, dx1_ref,
             w3a_ref, w3b_ref, b3_ref, g3_ref, be3_ref,
             w4_ref, b4_ref, g4_ref, be4_ref,
             hn_ref, xn_ref):
    h = h_ref[...]
    mi = mi0_ref[0] + mi1_ref[0]
    q = (jnp.dot(h, w3a_ref[...], preferred_element_type=jnp.float32)
         + jnp.dot(mi, w3b_ref[...], preferred_element_type=jnp.float32)
         + b3_ref[...])
    q = _silu(_ln(q, g3_ref[...], be3_ref[...]))
    ph = jnp.dot(q, w4_ref[...], preferred_element_type=jnp.float32) + b4_ref[...]
    ph = _ln(ph, g4_ref[...], be4_ref[...])
    hn_ref[...] = h + ph
    xn_ref[...] = x4_ref[...] + dx0_ref[0, :, :4] + dx1_ref[0, :, :4]


def _node_mlp(h, om, ox, x4,
              w3a, w3b, b3, g3, be3, w4, b4, g4, be4, bn=1000):
    n, d = h.shape
    grid = (n // bn,)
    col1 = lambda i: (i, 0)
    full = lambda i: (0, 0)
    spec_dd = pl.BlockSpec((d, d), full)
    spec_1d = pl.BlockSpec((1, d), full)
    return pl.pallas_call(
        _k5_body,
        grid=grid,
        in_specs=[
            pl.BlockSpec((bn, d), col1),
            pl.BlockSpec((1, bn, d), lambda i: (0, i, 0)),
            pl.BlockSpec((1, bn, d), lambda i: (1, i, 0)),
            pl.BlockSpec((bn, 4), col1),
            pl.BlockSpec((1, bn, d), lambda i: (0, i, 0)),
            pl.BlockSpec((1, bn, d), lambda i: (1, i, 0)),
            spec_dd, spec_dd, spec_1d, spec_1d, spec_1d,
            spec_dd, spec_1d, spec_1d, spec_1d,
        ],
        out_specs=[
            pl.BlockSpec((bn, d), col1),
            pl.BlockSpec((bn, 4), col1),
        ],
        out_shape=[
            jax.ShapeDtypeStruct((n, d), jnp.float32),
            jax.ShapeDtypeStruct((n, 4), jnp.float32),
        ],
    )(h, om, om, x4, ox, ox, w3a, w3b, b3, g3, be3, w4, b4, g4, be4)


def kernel(x, h, edge_index, local_frames, batch,
           W1, b1, g1, be1, W2, b2, g2, be2, Wg,
           W3, b3, g3, be3, W4, b4, g4, be4,
           W5, b5, g5, be5, W6, b6):
    n, din = h.shape
    e = edge_index.shape[1]
    row = edge_index[0]
    col = edge_index[1]

    w1a = W1[:din]
    w1b = W1[din:2 * din]
    w1c = W1[2 * din:2 * din + 1]
    w1d = W1[2 * din + 1:2 * din + 2]

    A, B = _node_proj(h, w1a, w1b)

    x4 = jnp.concatenate([x, jnp.zeros((n, 1), jnp.float32)], axis=-1)
    p, s = _sc_gather(A, B, x4.reshape(-1), col, row)
    chunk = 80
    nch = e // _NW // chunk
    S = s.reshape(_NW, nch, 5, chunk).transpose(2, 0, 1, 3).reshape(5, e)
    rx = S[0].reshape(e, 1)
    ry = S[1].reshape(e, 1)
    rz = S[2].reshape(e, 1)
    r2 = S[3].reshape(e, 1)
    pot = S[4].reshape(e, 1)

    mij, dx4 = _edge_mlp(
        p, r2, pot, rx, ry, rz,
        w1c, w1d, b1.reshape(1, -1), g1.reshape(1, -1), be1.reshape(1, -1),
        W2, b2.reshape(1, -1), g2.reshape(1, -1), be2.reshape(1, -1), Wg,
        W5, b5.reshape(1, -1), g5.reshape(1, -1), be5.reshape(1, -1),
        W6, b6.reshape(1, 1))

    npad = ((n + 8 * _NS - 1) // (8 * _NS)) * (8 * _NS)
    om, ox = _sc_scatter(mij, dx4, col, row, npad)

    hn, xn4 = _node_mlp(
        h, om, ox, x4,
        W3[:din], W3[din:], b3.reshape(1, -1), g3.reshape(1, -1),
        be3.reshape(1, -1), W4, b4.reshape(1, -1), g4.reshape(1, -1),
        be4.reshape(1, -1))

    return (xn4[:, :3], hn)


# speedup vs baseline: 4.6515x; 1.4813x over previous
import functools

import jax
import jax.numpy as jnp
from jax import lax
from jax.experimental import pallas as pl
from jax.experimental.pallas import tpu as pltpu
from jax.experimental.pallas import tpu_sc as plsc

_EPS = 1e-5
_NC, _NS, _L = 2, 16, 16
_NW = _NC * _NS


def _ln(t, g, b):
    mu = jnp.mean(t, axis=-1, keepdims=True)
    var = jnp.mean((t - mu) ** 2, axis=-1, keepdims=True)
    return (t - mu) * lax.rsqrt(var + _EPS) * g + b


def _silu(t):
    return t * jax.nn.sigmoid(t)


def _k1_body(h_ref, wa_ref, wb_ref, a_ref, b_ref):
    h = h_ref[...]
    a_ref[...] = jnp.dot(h, wa_ref[...], preferred_element_type=jnp.float32)
    b_ref[...] = jnp.dot(h, wb_ref[...], preferred_element_type=jnp.float32)


def _node_proj(h, wa, wb, bn=1000):
    n, d = h.shape
    grid = (n // bn,)
    return pl.pallas_call(
        _k1_body,
        grid=grid,
        in_specs=[
            pl.BlockSpec((bn, d), lambda i: (i, 0)),
            pl.BlockSpec((d, d), lambda i: (0, 0)),
            pl.BlockSpec((d, d), lambda i: (0, 0)),
        ],
        out_specs=[
            pl.BlockSpec((bn, d), lambda i: (i, 0)),
            pl.BlockSpec((bn, d), lambda i: (i, 0)),
        ],
        out_shape=[
            jax.ShapeDtypeStruct((n, d), jnp.float32),
            jax.ShapeDtypeStruct((n, d), jnp.float32),
        ],
    )(h, wa, wb)


def _sc_gather(A, B, x4flat, col, row, w1c, w1d, b1, chunk=80):
    n, d = A.shape
    e = col.shape[0]
    ept = e // _NW
    nchunks = ept // chunk
    npairs = (nchunks - 1) // 2
    assert nchunks == 2 * npairs + 1
    sb = 3 * chunk
    mesh = plsc.VectorSubcoreMesh(core_axis_name="c", subcore_axis_name="s")

    @functools.partial(
        pl.kernel, mesh=mesh,
        out_type=(
            jax.ShapeDtypeStruct((e, d), jnp.float32),
            jax.ShapeDtypeStruct((3 * e,), jnp.float32),
        ),
        compiler_params=pltpu.CompilerParams(needs_layout_passes=False),
        scratch_types=[
            pltpu.VMEM((chunk,), jnp.int32),
            pltpu.VMEM((chunk,), jnp.int32),
            pltpu.VMEM((chunk,), jnp.int32),
            pltpu.VMEM((chunk,), jnp.int32),
            pltpu.VMEM((chunk, d), jnp.float32),
            pltpu.VMEM((chunk, d), jnp.float32),
            pltpu.VMEM((chunk, d), jnp.float32),
            pltpu.VMEM((chunk, d), jnp.float32),
            pltpu.VMEM((chunk, d), jnp.float32),
            pltpu.VMEM((chunk, d), jnp.float32),
            pltpu.VMEM((n * 4,), jnp.float32),
            pltpu.VMEM((sb,), jnp.float32),
            pltpu.VMEM((sb,), jnp.float32),
            pltpu.VMEM((chunk,), jnp.float32),
            pltpu.VMEM((chunk,), jnp.float32),
            pltpu.VMEM((d,), jnp.float32),
            pltpu.VMEM((d,), jnp.float32),
            pltpu.VMEM((d,), jnp.float32),
            pltpu.SemaphoreType.DMA,
            pltpu.SemaphoreType.DMA,
            pltpu.SemaphoreType.DMA,
            pltpu.SemaphoreType.DMA,
        ],
    )
    def k(a_hbm, b_hbm, x_hbm, col_hbm, row_hbm, w1c_hbm, w1d_hbm, b1_hbm,
          p_hbm, s_hbm,
          idxc0, idxr0, idxc1, idxr1, abuf0, bbuf0, abuf1, bbuf1,
          pbuf0, pbuf1, xtab, sbuf0, sbuf1, r2b, potb, wcb, wdb, bbb,
          sem0, sem1, osem0, osem1):
        wid = lax.axis_index("s") * _NC + lax.axis_index("c")
        pltpu.sync_copy(x_hbm, xtab)
        pltpu.sync_copy(w1c_hbm, wcb)
        pltpu.sync_copy(w1d_hbm, wdb)
        pltpu.sync_copy(b1_hbm, bbb)

        def fire(ci, idxc, idxr, abuf, bbuf, sem):
            base = wid * ept + ci * chunk
            pltpu.sync_copy(col_hbm.at[pl.ds(base, chunk)], idxc)
            pltpu.sync_copy(row_hbm.at[pl.ds(base, chunk)], idxr)
            pltpu.async_copy(a_hbm.at[idxc], abuf, sem)
            pltpu.async_copy(b_hbm.at[idxr], bbuf, sem)

        def drain(idxc, idxr, abuf, bbuf, sem):
            pltpu.make_async_copy(a_hbm.at[idxc], abuf, sem).wait()
            pltpu.make_async_copy(b_hbm.at[idxr], bbuf, sem).wait()

        def compute_out(ci, idxc, idxr, abuf, bbuf, pbuf, sbuf, osem,
                        drain_out):
            base = wid * ept + ci * chunk
            soff = (wid * nchunks + ci) * sb
            if drain_out:
                pltpu.make_async_copy(pbuf, p_hbm.at[pl.ds(base, chunk)],
                                      osem).wait()
                pltpu.make_async_copy(sbuf, s_hbm.at[pl.ds(soff, sb)],
                                      osem).wait()

            for v in range(chunk // _L):
                s = pl.ds(v * _L, _L)
                cv = idxc[s] * 4
                rv = idxr[s] * 4
                dxc = plsc.load_gather(xtab, [cv]) - plsc.load_gather(xtab, [rv])
                dyc = (plsc.load_gather(xtab, [cv + 1])
                       - plsc.load_gather(xtab, [rv + 1]))
                dzc = (plsc.load_gather(xtab, [cv + 2])
                       - plsc.load_gather(xtab, [rv + 2]))
                r2v = dxc * dxc + dyc * dyc + dzc * dzc
                sbuf[pl.ds(v * _L, _L)] = dxc
                sbuf[pl.ds(chunk + v * _L, _L)] = dyc
                sbuf[pl.ds(2 * chunk + v * _L, _L)] = dzc
                r2b[s] = r2v
                potb[s] = 1.0 / (r2v + 1e-6)

            @plsc.parallel_loop(0, chunk, unroll=2)
            def add_body(i):
                r2s = plsc.load_gather(r2b, [jnp.full((_L,), i, jnp.int32)])
                pots = plsc.load_gather(potb, [jnp.full((_L,), i, jnp.int32)])
                for j in range(d // _L):
                    s = pl.ds(j * _L, _L)
                    pbuf[i, s] = (abuf[i, s] + bbuf[i, s] + bbb[s]
                                  + r2s * wcb[s] + pots * wdb[s])

            pltpu.async_copy(pbuf, p_hbm.at[pl.ds(base, chunk)], osem)
            pltpu.async_copy(sbuf, s_hbm.at[pl.ds(soff, sb)], osem)

        fire(0, idxc0, idxr0, abuf0, bbuf0, sem0)
        fire(1, idxc1, idxr1, abuf1, bbuf1, sem1)
        drain(idxc0, idxr0, abuf0, bbuf0, sem0)
        compute_out(0, idxc0, idxr0, abuf0, bbuf0, pbuf0, sbuf0, osem0, False)
        fire(2, idxc0, idxr0, abuf0, bbuf0, sem0)
        drain(idxc1, idxr1, abuf1, bbuf1, sem1)
        compute_out(1, idxc1, idxr1, abuf1, bbuf1, pbuf1, sbuf1, osem1, False)

        def pair_body(i, _):
            c0 = 2 * i + 2
            fire(c0 + 1, idxc1, idxr1, abuf1, bbuf1, sem1)
            drain(idxc0, idxr0, abuf0, bbuf0, sem0)
            compute_out(c0, idxc0, idxr0, abuf0, bbuf0, pbuf0, sbuf0, osem0,
                        True)
            fire(c0 + 2, idxc0, idxr0, abuf0, bbuf0, sem0)
            drain(idxc1, idxr1, abuf1, bbuf1, sem1)
            compute_out(c0 + 1, idxc1, idxr1, abuf1, bbuf1, pbuf1, sbuf1,
                        osem1, True)
            return 0

        lax.fori_loop(0, npairs - 1, pair_body, 0)
        drain(idxc0, idxr0, abuf0, bbuf0, sem0)
        compute_out(nchunks - 1, idxc0, idxr0, abuf0, bbuf0, pbuf0, sbuf0,
                    osem0, True)
        base_l = wid * ept + (nchunks - 1) * chunk
        soff_l = (wid * nchunks + (nchunks - 1)) * sb
        pltpu.make_async_copy(pbuf0, p_hbm.at[pl.ds(base_l, chunk)],
                              osem0).wait()
        pltpu.make_async_copy(sbuf0, s_hbm.at[pl.ds(soff_l, sb)], osem0).wait()
        pltpu.make_async_copy(pbuf1, p_hbm.at[pl.ds(base_l, chunk)],
                              osem1).wait()
        pltpu.make_async_copy(sbuf1, s_hbm.at[pl.ds(soff_l, sb)], osem1).wait()

    return k(A, B, x4flat, col, row, w1c, w1d, b1)


def _k3_body(p_ref, g1_ref, be1_ref,
             w2_ref, b2_ref, g2_ref, be2_ref, wg_ref,
             w5_ref, b5_ref, g5_ref, be5_ref, w6_ref, b6_ref,
             mij_ref, w_ref):
    pre1 = p_ref[...]
    u = _silu(_ln(pre1, g1_ref[...], be1_ref[...]))
    m = jnp.dot(u, w2_ref[...], preferred_element_type=jnp.float32) + b2_ref[...]
    m = _silu(_ln(m, g2_ref[...], be2_ref[...]))
    gate = jax.nn.sigmoid(jnp.dot(m, wg_ref[...], preferred_element_type=jnp.float32))
    mij = m * gate
    mij_ref[...] = mij
    t = jnp.dot(mij, w5_ref[...], preferred_element_type=jnp.float32) + b5_ref[...]
    t = _silu(_ln(t, g5_ref[...], be5_ref[...]))
    w_ref[...] = jax.nn.sigmoid(
        jnp.dot(t, w6_ref[...], preferred_element_type=jnp.float32) + b6_ref[...])


def _edge_mlp(p, g1, be1, w2, b2, g2, be2, wg,
              w5, b5, g5, be5, w6, b6, be_blk=2000):
    e, d = p.shape
    grid = (e // be_blk,)
    col1 = lambda i: (i, 0)
    full = lambda i: (0, 0)
    spec_dd = pl.BlockSpec((d, d), full)
    spec_1d = pl.BlockSpec((1, d), full)
    spec_d1 = pl.BlockSpec((d, 1), full)
    spec_11 = pl.BlockSpec((1, 1), full)
    return pl.pallas_call(
        _k3_body,
        grid=grid,
        in_specs=[
            pl.BlockSpec((be_blk, d), col1),
            spec_1d, spec_1d,
            spec_dd, spec_1d, spec_1d, spec_1d, spec_d1,
            spec_dd, spec_1d, spec_1d, spec_1d, spec_d1, spec_11,
        ],
        out_specs=[
            pl.BlockSpec((be_blk, d), col1),
            pl.BlockSpec((be_blk, 1), col1),
        ],
        out_shape=[
            jax.ShapeDtypeStruct((e, d), jnp.float32),
            jax.ShapeDtypeStruct((e, 1), jnp.float32),
        ],
    )(p, g1, be1, w2, b2, g2, be2, wg, w5, b5, g5, be5, w6, b6)


def _sc_scatter(mij, s3, wgt, col, row, npad, chunk=80):
    e, d = mij.shape
    sb = 3 * chunk
    ept = e // _NW
    nchunks = ept // chunk
    npt = npad // _NS
    mesh = plsc.VectorSubcoreMesh(core_axis_name="c", subcore_axis_name="s")

    npairs = (nchunks - 1) // 2
    assert nchunks == 2 * npairs + 1

    @functools.partial(
        pl.kernel, mesh=mesh,
        out_type=(
            jax.ShapeDtypeStruct((_NC, npad, d), jnp.float32),
            jax.ShapeDtypeStruct((_NC, npad, d), jnp.float32),
        ),
        compiler_params=pltpu.CompilerParams(needs_layout_passes=False),
        scratch_types=[
            pltpu.VMEM((chunk,), jnp.int32),
            pltpu.VMEM((chunk,), jnp.int32),
            pltpu.VMEM((chunk, d), jnp.float32),
            pltpu.VMEM((chunk, d), jnp.float32),
            pltpu.VMEM((sb,), jnp.float32),
            pltpu.VMEM((sb,), jnp.float32),
            pltpu.VMEM((chunk,), jnp.float32),
            pltpu.VMEM((chunk,), jnp.float32),
            pltpu.VMEM((chunk, d), jnp.float32),
            pltpu.VMEM_SHARED((npad, d), jnp.float32),
            pltpu.SemaphoreType.DMA,
            pltpu.SemaphoreType.DMA,
        ],
    )
    def k(m_hbm, s3_hbm, w_hbm, col_hbm, row_hbm, zm_hbm, om_hbm, ox_hbm,
          idx0, idx1, mbuf0, mbuf1, s3buf0, s3buf1, wbuf0, wbuf1,
          xpand, sm, sem0, sem1):
        cid = lax.axis_index("c")
        sid = lax.axis_index("s")
        wid = sid * _NC + cid
        nbase = sid * npt

        pltpu.sync_copy(zm_hbm.at[pl.ds(nbase, npt)], sm.at[pl.ds(nbase, npt)])
        zv = jnp.zeros((_L,), jnp.float32)

        def zexp_body(i, _):
            for j in range(d // _L):
                xpand[i, pl.ds(j * _L, _L)] = zv
            return 0
        lax.fori_loop(0, chunk, zexp_body, 0)
        plsc.subcore_barrier()

        def m_fire(ci, idx, mbuf, sem):
            base = wid * ept + ci * chunk
            pltpu.sync_copy(col_hbm.at[pl.ds(base, chunk)], idx)
            pltpu.async_copy(m_hbm.at[pl.ds(base, chunk)], mbuf, sem)

        def m_scat(ci, idx, mbuf, sem):
            base = wid * ept + ci * chunk
            pltpu.make_async_copy(
                m_hbm.at[pl.ds(base, chunk)], mbuf, sem).wait()
            pltpu.sync_copy(mbuf, sm.at[idx], add=True)

        m_fire(0, idx0, mbuf0, sem0)

        def m_pair(i, _):
            c0 = 2 * i
            m_fire(c0 + 1, idx1, mbuf1, sem1)
            m_scat(c0, idx0, mbuf0, sem0)
            m_fire(c0 + 2, idx0, mbuf0, sem0)
            m_scat(c0 + 1, idx1, mbuf1, sem1)
            return 0

        lax.fori_loop(0, npairs, m_pair, 0)
        m_scat(nchunks - 1, idx0, mbuf0, sem0)

        plsc.subcore_barrier()
        pltpu.sync_copy(sm.at[pl.ds(nbase, npt)], om_hbm.at[cid, pl.ds(nbase, npt)])
        plsc.subcore_barrier()

        pltpu.sync_copy(zm_hbm.at[pl.ds(nbase, npt)], sm.at[pl.ds(nbase, npt)])
        plsc.subcore_barrier()

        lane = lax.iota(jnp.int32, _L)

        def dx_fire(ci, idx, s3buf, wbuf, sem):
            base = wid * ept + ci * chunk
            soff = (wid * nchunks + ci) * sb
            pltpu.sync_copy(row_hbm.at[pl.ds(base, chunk)], idx)
            pltpu.async_copy(s3_hbm.at[pl.ds(soff, sb)], s3buf, sem)
            pltpu.async_copy(w_hbm.at[pl.ds(base, chunk)], wbuf, sem)

        def dx_scat(ci, idx, s3buf, wbuf, sem):
            base = wid * ept + ci * chunk
            soff = (wid * nchunks + ci) * sb
            pltpu.make_async_copy(
                s3_hbm.at[pl.ds(soff, sb)], s3buf, sem).wait()
            pltpu.make_async_copy(
                w_hbm.at[pl.ds(base, chunk)], wbuf, sem).wait()
            for v in range(chunk // _L):
                wv = wbuf[pl.ds(v * _L, _L)]
                rows = v * _L + lane
                for c in range(3):
                    vals = s3buf[pl.ds(c * chunk + v * _L, _L)] * wv
                    cols = jnp.full((_L,), c, jnp.int32)
                    plsc.store_scatter(xpand, [rows, cols], vals)
            pltpu.sync_copy(xpand, sm.at[idx], add=True)

        dx_fire(0, idx0, s3buf0, wbuf0, sem0)

        def dx_pair(i, _):
            c0 = 2 * i
            dx_fire(c0 + 1, idx1, s3buf1, wbuf1, sem1)
            dx_scat(c0, idx0, s3buf0, wbuf0, sem0)
            dx_fire(c0 + 2, idx0, s3buf0, wbuf0, sem0)
            dx_scat(c0 + 1, idx1, s3buf1, wbuf1, sem1)
            return 0

        lax.fori_loop(0, npairs, dx_pair, 0)
        dx_scat(nchunks - 1, idx0, s3buf0, wbuf0, sem0)

        plsc.subcore_barrier()
        pltpu.sync_copy(sm.at[pl.ds(nbase, npt)], ox_hbm.at[cid, pl.ds(nbase, npt)])

    return k(mij, s3, wgt, col, row, jnp.zeros((npad, d), jnp.float32))


def _k5_body(h_ref, mi0_ref, mi1_ref, x4_ref, dx0_ref, dx1_ref,
             w3a_ref, w3b_ref, b3_ref, g3_ref, be3_ref,
             w4_ref, b4_ref, g4_ref, be4_ref,
             hn_ref, xn_ref):
    h = h_ref[...]
    mi = mi0_ref[0] + mi1_ref[0]
    q = (jnp.dot(h, w3a_ref[...], preferred_element_type=jnp.float32)
         + jnp.dot(mi, w3b_ref[...], preferred_element_type=jnp.float32)
         + b3_ref[...])
    q = _silu(_ln(q, g3_ref[...], be3_ref[...]))
    ph = jnp.dot(q, w4_ref[...], preferred_element_type=jnp.float32) + b4_ref[...]
    ph = _ln(ph, g4_ref[...], be4_ref[...])
    hn_ref[...] = h + ph
    xn_ref[...] = x4_ref[...] + dx0_ref[0, :, :4] + dx1_ref[0, :, :4]


def _node_mlp(h, om, ox, x4,
              w3a, w3b, b3, g3, be3, w4, b4, g4, be4, bn=1000):
    n, d = h.shape
    grid = (n // bn,)
    col1 = lambda i: (i, 0)
    full = lambda i: (0, 0)
    spec_dd = pl.BlockSpec((d, d), full)
    spec_1d = pl.BlockSpec((1, d), full)
    return pl.pallas_call(
        _k5_body,
        grid=grid,
        in_specs=[
            pl.BlockSpec((bn, d), col1),
            pl.BlockSpec((1, bn, d), lambda i: (0, i, 0)),
            pl.BlockSpec((1, bn, d), lambda i: (1, i, 0)),
            pl.BlockSpec((bn, 4), col1),
            pl.BlockSpec((1, bn, d), lambda i: (0, i, 0)),
            pl.BlockSpec((1, bn, d), lambda i: (1, i, 0)),
            spec_dd, spec_dd, spec_1d, spec_1d, spec_1d,
            spec_dd, spec_1d, spec_1d, spec_1d,
        ],
        out_specs=[
            pl.BlockSpec((bn, d), col1),
            pl.BlockSpec((bn, 4), col1),
        ],
        out_shape=[
            jax.ShapeDtypeStruct((n, d), jnp.float32),
            jax.ShapeDtypeStruct((n, 4), jnp.float32),
        ],
    )(h, om, om, x4, ox, ox, w3a, w3b, b3, g3, be3, w4, b4, g4, be4)


def kernel(x, h, edge_index, local_frames, batch,
           W1, b1, g1, be1, W2, b2, g2, be2, Wg,
           W3, b3, g3, be3, W4, b4, g4, be4,
           W5, b5, g5, be5, W6, b6):
    n, din = h.shape
    e = edge_index.shape[1]
    row = edge_index[0]
    col = edge_index[1]

    w1a = W1[:din]
    w1b = W1[din:2 * din]
    w1c = W1[2 * din:2 * din + 1]
    w1d = W1[2 * din + 1:2 * din + 2]

    A, B = _node_proj(h, w1a, w1b)

    x4 = jnp.concatenate([x, jnp.zeros((n, 1), jnp.float32)], axis=-1)
    p, s = _sc_gather(A, B, x4.reshape(-1), col, row,
                      W1[2 * din], W1[2 * din + 1], b1)

    mij, w = _edge_mlp(
        p, g1.reshape(1, -1), be1.reshape(1, -1),
        W2, b2.reshape(1, -1), g2.reshape(1, -1), be2.reshape(1, -1), Wg,
        W5, b5.reshape(1, -1), g5.reshape(1, -1), be5.reshape(1, -1),
        W6, b6.reshape(1, 1))

    npad = ((n + 8 * _NS - 1) // (8 * _NS)) * (8 * _NS)
    om, ox = _sc_scatter(mij, s, w.reshape(-1), col, row, npad)

    hn, xn4 = _node_mlp(
        h, om, ox, x4,
        W3[:din], W3[din:], b3.reshape(1, -1), g3.reshape(1, -1),
        be3.reshape(1, -1), W4, b4.reshape(1, -1), g4.reshape(1, -1),
        be4.reshape(1, -1))

    return (xn4[:, :3], hn)
